# Initial kernel scaffold; baseline (speedup 1.0000x reference)
import jax
import jax.numpy as jnp
from jax.experimental import pallas as pl


def kernel(x, edge_index, W1, b1, W2, b2):
    raise NotImplementedError("write your pallas kernel here")


# speedup vs baseline: 13.4843x; 13.4843x over previous
import functools

import jax
import jax.numpy as jnp
from jax import lax
from jax.experimental import pallas as pl
from jax.experimental.pallas import tpu as pltpu
from jax.experimental.pallas import tpu_sc as plsc

NC = 2
NS = 16
NW = NC * NS
K = 128
DEG_W = 16

_mesh = lambda: plsc.VectorSubcoreMesh(core_axis_name="c", subcore_axis_name="s")
_SC_PARAMS = pltpu.CompilerParams(use_tc_tiling_on_sc=False)


def _sc_degree(cols3, n_pad, cpw):
    rpw = n_pad // NS
    nzc, rem = divmod(rpw, K)

    @functools.partial(
        pl.kernel,
        out_type=jax.ShapeDtypeStruct((NC, n_pad, DEG_W), jnp.float32),
        mesh=_mesh(),
        compiler_params=_SC_PARAMS,
        scratch_types=[
            pltpu.VMEM((cpw, K), jnp.int32),
            pltpu.VMEM((K, DEG_W), jnp.float32),
            pltpu.VMEM((K, DEG_W), jnp.float32),
            pltpu.VMEM_SHARED((n_pad, DEG_W), jnp.float32),
        ],
    )
    def kern(cols_hbm, ones_hbm, z_hbm, out_hbm, cidx, onesb, zb, acc):
        c = lax.axis_index("c")
        s = lax.axis_index("s")
        wid = s * NC + c
        pltpu.sync_copy(cols_hbm.at[wid], cidx)
        pltpu.sync_copy(ones_hbm, onesb)
        pltpu.sync_copy(z_hbm, zb)
        base = s * rpw

        @pl.loop(0, nzc)
        def _(i):
            pltpu.sync_copy(zb, acc.at[pl.ds(base + i * K, K)])

        if rem:
            pltpu.sync_copy(zb.at[pl.ds(0, rem)], acc.at[pl.ds(base + nzc * K, rem)])
        plsc.subcore_barrier()

        @pl.loop(0, cpw)
        def _(i):
            pltpu.sync_copy(onesb, acc.at[cidx.at[i]], add=True)

        plsc.subcore_barrier()
        pltpu.sync_copy(acc.at[pl.ds(base, rpw)], out_hbm.at[c, pl.ds(base, rpw)])

    ones = jnp.ones((K, DEG_W), jnp.float32)
    zeros = jnp.zeros((K, DEG_W), jnp.float32)
    return kern(cols3, ones, zeros)


def _sc_scatter_sum(vals, rows3, cols3, n_pad, d, cpw):
    rpw = n_pad // NS
    nzc, rem = divmod(rpw, K)

    @functools.partial(
        pl.kernel,
        out_type=jax.ShapeDtypeStruct((NC, n_pad, d), jnp.float32),
        mesh=_mesh(),
        compiler_params=_SC_PARAMS,
        scratch_types=[
            pltpu.VMEM((cpw, K), jnp.int32),
            pltpu.VMEM((cpw, K), jnp.int32),
            pltpu.VMEM((K, d), jnp.float32),
            pltpu.VMEM((K, d), jnp.float32),
            pltpu.VMEM_SHARED((n_pad, d), jnp.float32),
            pltpu.SemaphoreType.DMA,
            pltpu.SemaphoreType.DMA,
        ],
    )
    def kern(vals_hbm, rows_hbm, cols_hbm, z_hbm, out_hbm,
             ridx, cidx, buf0, buf1, acc, sem0, sem1):
        c = lax.axis_index("c")
        s = lax.axis_index("s")
        wid = s * NC + c
        pltpu.sync_copy(rows_hbm.at[wid], ridx)
        pltpu.sync_copy(cols_hbm.at[wid], cidx)
        pltpu.sync_copy(z_hbm, buf0)
        base = s * rpw

        @pl.loop(0, nzc)
        def _(i):
            pltpu.sync_copy(buf0, acc.at[pl.ds(base + i * K, K)])

        if rem:
            pltpu.sync_copy(buf0.at[pl.ds(0, rem)], acc.at[pl.ds(base + nzc * K, rem)])
        pltpu.async_copy(vals_hbm.at[ridx.at[0]], buf0, sem0)
        plsc.subcore_barrier()

        @pl.loop(0, cpw // 2)
        def _(it):
            i0 = 2 * it
            pltpu.make_async_copy(vals_hbm.at[ridx.at[i0]], buf0, sem0).wait()
            pltpu.async_copy(vals_hbm.at[ridx.at[i0 + 1]], buf1, sem1)
            pltpu.sync_copy(buf0, acc.at[cidx.at[i0]], add=True)
            pltpu.make_async_copy(vals_hbm.at[ridx.at[i0 + 1]], buf1, sem1).wait()

            @pl.when(i0 + 2 < cpw)
            def _():
                pltpu.async_copy(vals_hbm.at[ridx.at[i0 + 2]], buf0, sem0)

            pltpu.sync_copy(buf1, acc.at[cidx.at[i0 + 1]], add=True)

        plsc.subcore_barrier()
        pltpu.sync_copy(acc.at[pl.ds(base, rpw)], out_hbm.at[c, pl.ds(base, rpw)])

    zeros = jnp.zeros((K, d), jnp.float32)
    return kern(vals, rows3, cols3, zeros)


def _tc_scale(x, deg0, deg1, tm):
    n, d = x.shape
    dh = d // 2

    def body(x_ref, d0_ref, d1_ref, oa_ref, ob_ref):
        dinv = lax.rsqrt(d0_ref[...] + d1_ref[...] + 1.0)
        xs = dinv * x_ref[...]
        oa_ref[...] = xs[:, :dh]
        ob_ref[...] = xs[:, dh:]

    return pl.pallas_call(
        body,
        grid=(n // tm,),
        in_specs=[
            pl.BlockSpec((tm, d), lambda i: (i, 0)),
            pl.BlockSpec((tm, 1), lambda i: (i, 0)),
            pl.BlockSpec((tm, 1), lambda i: (i, 0)),
        ],
        out_specs=[
            pl.BlockSpec((tm, dh), lambda i: (i, 0)),
            pl.BlockSpec((tm, dh), lambda i: (i, 0)),
        ],
        out_shape=[
            jax.ShapeDtypeStruct((n, dh), jnp.float32),
            jax.ShapeDtypeStruct((n, dh), jnp.float32),
        ],
    )(x, deg0, deg1)


def _tc_dense(pa0, pa1, pb0, pb1, x, deg0, deg1, W1, b1, W2p, tm):
    n, d_in = x.shape
    dh = d_in // 2
    d_h = W1.shape[1]
    d_o = W2p.shape[1]

    def body(pa0_ref, pa1_ref, pb0_ref, pb1_ref, x_ref, d0_ref, d1_ref,
             w1_ref, b1_ref, w2_ref, h_ref, ts_ref):
        dinv = lax.rsqrt(d0_ref[...] + d1_ref[...] + 1.0)
        d2 = dinv * dinv
        x_blk = x_ref[...]
        agg_a = dinv * (pa0_ref[...] + pa1_ref[...]) + d2 * x_blk[:, :dh]
        agg_b = dinv * (pb0_ref[...] + pb1_ref[...]) + d2 * x_blk[:, dh:]
        w1 = w1_ref[...]
        h = (jnp.dot(agg_a, w1[:dh], preferred_element_type=jnp.float32)
             + jnp.dot(agg_b, w1[dh:], preferred_element_type=jnp.float32))
        h = jnp.maximum(h + b1_ref[...], 0.0)
        h_ref[...] = h
        t = jnp.dot(h, w2_ref[...], preferred_element_type=jnp.float32)
        ts_ref[...] = dinv * t

    return pl.pallas_call(
        body,
        grid=(n // tm,),
        in_specs=[
            pl.BlockSpec((tm, dh), lambda i: (i, 0)),
            pl.BlockSpec((tm, dh), lambda i: (i, 0)),
            pl.BlockSpec((tm, dh), lambda i: (i, 0)),
            pl.BlockSpec((tm, dh), lambda i: (i, 0)),
            pl.BlockSpec((tm, d_in), lambda i: (i, 0)),
            pl.BlockSpec((tm, 1), lambda i: (i, 0)),
            pl.BlockSpec((tm, 1), lambda i: (i, 0)),
            pl.BlockSpec((d_in, d_h), lambda i: (0, 0)),
            pl.BlockSpec((1, d_h), lambda i: (0, 0)),
            pl.BlockSpec((d_h, d_o), lambda i: (0, 0)),
        ],
        out_specs=[
            pl.BlockSpec((tm, d_h), lambda i: (i, 0)),
            pl.BlockSpec((tm, d_o), lambda i: (i, 0)),
        ],
        out_shape=[
            jax.ShapeDtypeStruct((n, d_h), jnp.float32),
            jax.ShapeDtypeStruct((n, d_o), jnp.float32),
        ],
    )(pa0, pa1, pb0, pb1, x, deg0, deg1, W1, b1, W2p)


def _tc_head(q0, q1, ts, deg0, deg1, b2p, tm):
    n, d = ts.shape

    def body(q0_ref, q1_ref, ts_ref, d0_ref, d1_ref, b2_ref, o_ref):
        dinv = lax.rsqrt(d0_ref[...] + d1_ref[...] + 1.0)
        z = dinv * (q0_ref[...] + q1_ref[...] + ts_ref[...]) + b2_ref[...]
        o_ref[...] = jnp.maximum(z, 0.0) + jnp.log1p(jnp.exp(-jnp.abs(z)))

    return pl.pallas_call(
        body,
        grid=(n // tm,),
        in_specs=[
            pl.BlockSpec((tm, d), lambda i: (i, 0)),
            pl.BlockSpec((tm, d), lambda i: (i, 0)),
            pl.BlockSpec((tm, d), lambda i: (i, 0)),
            pl.BlockSpec((tm, 1), lambda i: (i, 0)),
            pl.BlockSpec((tm, 1), lambda i: (i, 0)),
            pl.BlockSpec((1, d), lambda i: (0, 0)),
        ],
        out_specs=pl.BlockSpec((tm, d), lambda i: (i, 0)),
        out_shape=jax.ShapeDtypeStruct((n, d), jnp.float32),
    )(q0, q1, ts, deg0, deg1, b2p)


def kernel(x, edge_index, W1, b1, W2, b2):
    n = x.shape[0]
    e = edge_index.shape[1]

    n_pad = 8 * NS * -(-(n + DEG_W) // (8 * NS))
    cpw = -(-e // (NW * K))
    cpw += cpw % 2
    e_pad = NW * cpw * K

    row = edge_index[0].astype(jnp.int32)
    col = edge_index[1].astype(jnp.int32)
    pad = e_pad - e
    prow = jnp.zeros((pad,), jnp.int32)
    pcol = n + (jnp.arange(pad, dtype=jnp.int32) % (n_pad - n))
    rows3 = jnp.concatenate([row, prow]).reshape(NW, cpw, K)
    cols3 = jnp.concatenate([col, pcol]).reshape(NW, cpw, K)

    tm = 2000 if n % 2000 == 0 else 8 * (n // 8)

    deg = _sc_degree(cols3, n_pad, cpw)
    deg0 = deg[0, :n, :1]
    deg1 = deg[1, :n, :1]

    xsa, xsb = _tc_scale(x, deg0, deg1, tm)
    dh = x.shape[1] // 2
    pa = _sc_scatter_sum(xsa, rows3, cols3, n_pad, dh, cpw)
    pb = _sc_scatter_sum(xsb, rows3, cols3, n_pad, dh, cpw)

    W2p = jnp.pad(W2, ((0, 0), (0, -W2.shape[1] % DEG_W)))
    b2p = jnp.pad(b2, (0, -b2.shape[0] % DEG_W)).reshape(1, -1)
    h, ts = _tc_dense(pa[0, :n], pa[1, :n], pb[0, :n], pb[1, :n],
                      x, deg0, deg1, W1, b1.reshape(1, -1), W2p, tm)

    q = _sc_scatter_sum(ts, rows3, cols3, n_pad, W2p.shape[1], cpw)
    ev = _tc_head(q[0, :n], q[1, :n], ts, deg0, deg1, b2p, tm)
    return ev[:, : W2.shape[1]], h


# speedup vs baseline: 15.3422x; 1.1378x over previous
import functools

import jax
import jax.numpy as jnp
from jax import lax
from jax.experimental import pallas as pl
from jax.experimental.pallas import tpu as pltpu
from jax.experimental.pallas import tpu_sc as plsc

NC = 2
NS = 16
NW = NC * NS
K = 128
DEG_W = 16

_mesh = lambda: plsc.VectorSubcoreMesh(core_axis_name="c", subcore_axis_name="s")
_SC_PARAMS = pltpu.CompilerParams(use_tc_tiling_on_sc=False)


def _sc_degree(cols3, n_pad, cpw):
    rpw = n_pad // NS
    nzc, rem = divmod(rpw, K)

    @functools.partial(
        pl.kernel,
        out_type=jax.ShapeDtypeStruct((NC, n_pad, DEG_W), jnp.float32),
        mesh=_mesh(),
        compiler_params=_SC_PARAMS,
        scratch_types=[
            pltpu.VMEM((cpw, K), jnp.int32),
            pltpu.VMEM((K, DEG_W), jnp.float32),
            pltpu.VMEM((K, DEG_W), jnp.float32),
            pltpu.VMEM_SHARED((n_pad, DEG_W), jnp.float32),
        ],
    )
    def kern(cols_hbm, ones_hbm, z_hbm, out_hbm, cidx, onesb, zb, acc):
        c = lax.axis_index("c")
        s = lax.axis_index("s")
        wid = s * NC + c
        pltpu.sync_copy(cols_hbm.at[wid], cidx)
        pltpu.sync_copy(ones_hbm, onesb)
        pltpu.sync_copy(z_hbm, zb)
        base = s * rpw

        @pl.loop(0, nzc)
        def _(i):
            pltpu.sync_copy(zb, acc.at[pl.ds(base + i * K, K)])

        if rem:
            pltpu.sync_copy(zb.at[pl.ds(0, rem)], acc.at[pl.ds(base + nzc * K, rem)])
        plsc.subcore_barrier()

        @pl.loop(0, cpw)
        def _(i):
            pltpu.sync_copy(onesb, acc.at[cidx.at[i]], add=True)

        plsc.subcore_barrier()
        pltpu.sync_copy(acc.at[pl.ds(base, rpw)], out_hbm.at[c, pl.ds(base, rpw)])

    ones = jnp.ones((K, DEG_W), jnp.float32)
    zeros = jnp.zeros((K, DEG_W), jnp.float32)
    return kern(cols3, ones, zeros)


def _sc_scatter_sum(vals, rows3, cols3, n_pad, d, cpw):
    rpw = n_pad // NS
    nzc, rem = divmod(rpw, K)

    @functools.partial(
        pl.kernel,
        out_type=jax.ShapeDtypeStruct((NC, n_pad, d), jnp.float32),
        mesh=_mesh(),
        compiler_params=_SC_PARAMS,
        scratch_types=[
            pltpu.VMEM((cpw, K), jnp.int32),
            pltpu.VMEM((cpw, K), jnp.int32),
            pltpu.VMEM((K, d), jnp.float32),
            pltpu.VMEM((K, d), jnp.float32),
            pltpu.VMEM_SHARED((n_pad, d), jnp.float32),
            pltpu.SemaphoreType.DMA,
            pltpu.SemaphoreType.DMA,
        ],
    )
    def kern(vals_hbm, rows_hbm, cols_hbm, z_hbm, out_hbm,
             ridx, cidx, buf0, buf1, acc, sem0, sem1):
        c = lax.axis_index("c")
        s = lax.axis_index("s")
        wid = s * NC + c
        pltpu.sync_copy(rows_hbm.at[wid], ridx)
        pltpu.sync_copy(cols_hbm.at[wid], cidx)
        pltpu.sync_copy(z_hbm, buf0)
        base = s * rpw

        @pl.loop(0, nzc)
        def _(i):
            pltpu.sync_copy(buf0, acc.at[pl.ds(base + i * K, K)])

        if rem:
            pltpu.sync_copy(buf0.at[pl.ds(0, rem)], acc.at[pl.ds(base + nzc * K, rem)])
        pltpu.async_copy(vals_hbm.at[ridx.at[0]], buf0, sem0)
        plsc.subcore_barrier()

        @pl.loop(0, cpw // 2)
        def _(it):
            i0 = 2 * it
            pltpu.make_async_copy(vals_hbm.at[ridx.at[i0]], buf0, sem0).wait()
            pltpu.async_copy(vals_hbm.at[ridx.at[i0 + 1]], buf1, sem1)
            pltpu.sync_copy(buf0, acc.at[cidx.at[i0]], add=True)
            pltpu.make_async_copy(vals_hbm.at[ridx.at[i0 + 1]], buf1, sem1).wait()

            @pl.when(i0 + 2 < cpw)
            def _():
                pltpu.async_copy(vals_hbm.at[ridx.at[i0 + 2]], buf0, sem0)

            pltpu.sync_copy(buf1, acc.at[cidx.at[i0 + 1]], add=True)

        plsc.subcore_barrier()
        pltpu.sync_copy(acc.at[pl.ds(base, rpw)], out_hbm.at[c, pl.ds(base, rpw)])

    zeros = jnp.zeros((K, d), jnp.float32)
    return kern(vals, rows3, cols3, zeros)


def _sc_scatter_sum_split(vals2, rows3, cols3, n_pad, dh, cpw2):
    rpw = n_pad // NS
    nzc, rem = divmod(rpw, K)

    @functools.partial(
        pl.kernel,
        out_type=jax.ShapeDtypeStruct((NC, n_pad, dh), jnp.float32),
        mesh=_mesh(),
        compiler_params=_SC_PARAMS,
        scratch_types=[
            pltpu.VMEM((cpw2, K), jnp.int32),
            pltpu.VMEM((cpw2, K), jnp.int32),
            pltpu.VMEM((K, dh), jnp.float32),
            pltpu.VMEM((K, dh), jnp.float32),
            pltpu.VMEM_SHARED((n_pad, dh), jnp.float32),
            pltpu.SemaphoreType.DMA,
            pltpu.SemaphoreType.DMA,
        ],
    )
    def kern(vals_hbm, rows_hbm, cols_hbm, z_hbm, out_hbm,
             ridx, cidx, buf0, buf1, acc, sem0, sem1):
        c = lax.axis_index("c")
        s = lax.axis_index("s")
        src = vals_hbm.at[c]
        pltpu.sync_copy(rows_hbm.at[s], ridx)
        pltpu.sync_copy(cols_hbm.at[s], cidx)
        pltpu.sync_copy(z_hbm, buf0)
        base = s * rpw

        @pl.loop(0, nzc)
        def _(i):
            pltpu.sync_copy(buf0, acc.at[pl.ds(base + i * K, K)])

        if rem:
            pltpu.sync_copy(buf0.at[pl.ds(0, rem)], acc.at[pl.ds(base + nzc * K, rem)])
        pltpu.async_copy(src.at[ridx.at[0]], buf0, sem0)
        plsc.subcore_barrier()

        @pl.loop(0, cpw2 // 2)
        def _(it):
            i0 = 2 * it
            pltpu.make_async_copy(src.at[ridx.at[i0]], buf0, sem0).wait()
            pltpu.async_copy(src.at[ridx.at[i0 + 1]], buf1, sem1)
            pltpu.sync_copy(buf0, acc.at[cidx.at[i0]], add=True)
            pltpu.make_async_copy(src.at[ridx.at[i0 + 1]], buf1, sem1).wait()

            @pl.when(i0 + 2 < cpw2)
            def _():
                pltpu.async_copy(src.at[ridx.at[i0 + 2]], buf0, sem0)

            pltpu.sync_copy(buf1, acc.at[cidx.at[i0 + 1]], add=True)

        plsc.subcore_barrier()
        pltpu.sync_copy(acc.at[pl.ds(base, rpw)], out_hbm.at[c, pl.ds(base, rpw)])

    zeros = jnp.zeros((K, dh), jnp.float32)
    return kern(vals2, rows3, cols3, zeros)


def _tc_scale(x, deg0, deg1, tm):
    n, d = x.shape
    dh = d // 2

    def body(x_ref, d0_ref, d1_ref, o_ref):
        dinv = lax.rsqrt(d0_ref[...] + d1_ref[...] + 1.0)
        xs = dinv * x_ref[...]
        o_ref[0] = xs[:, :dh]
        o_ref[1] = xs[:, dh:]

    return pl.pallas_call(
        body,
        grid=(n // tm,),
        in_specs=[
            pl.BlockSpec((tm, d), lambda i: (i, 0)),
            pl.BlockSpec((tm, 1), lambda i: (i, 0)),
            pl.BlockSpec((tm, 1), lambda i: (i, 0)),
        ],
        out_specs=pl.BlockSpec((2, tm, dh), lambda i: (0, i, 0)),
        out_shape=jax.ShapeDtypeStruct((2, n, dh), jnp.float32),
    )(x, deg0, deg1)


def _tc_dense(pa, pb, x, deg0, deg1, W1, b1, W2p, tm):
    n, d_in = x.shape
    dh = d_in // 2
    d_h = W1.shape[1]
    d_o = W2p.shape[1]

    def body(pa_ref, pb_ref, x_ref, d0_ref, d1_ref,
             w1_ref, b1_ref, w2_ref, h_ref, ts_ref):
        dinv = lax.rsqrt(d0_ref[...] + d1_ref[...] + 1.0)
        d2 = dinv * dinv
        x_blk = x_ref[...]
        agg_a = dinv * pa_ref[...] + d2 * x_blk[:, :dh]
        agg_b = dinv * pb_ref[...] + d2 * x_blk[:, dh:]
        w1 = w1_ref[...]
        h = (jnp.dot(agg_a, w1[:dh], preferred_element_type=jnp.float32)
             + jnp.dot(agg_b, w1[dh:], preferred_element_type=jnp.float32))
        h = jnp.maximum(h + b1_ref[...], 0.0)
        h_ref[...] = h
        t = jnp.dot(h, w2_ref[...], preferred_element_type=jnp.float32)
        ts_ref[...] = dinv * t

    return pl.pallas_call(
        body,
        grid=(n // tm,),
        in_specs=[
            pl.BlockSpec((tm, dh), lambda i: (i, 0)),
            pl.BlockSpec((tm, dh), lambda i: (i, 0)),
            pl.BlockSpec((tm, d_in), lambda i: (i, 0)),
            pl.BlockSpec((tm, 1), lambda i: (i, 0)),
            pl.BlockSpec((tm, 1), lambda i: (i, 0)),
            pl.BlockSpec((d_in, d_h), lambda i: (0, 0)),
            pl.BlockSpec((1, d_h), lambda i: (0, 0)),
            pl.BlockSpec((d_h, d_o), lambda i: (0, 0)),
        ],
        out_specs=[
            pl.BlockSpec((tm, d_h), lambda i: (i, 0)),
            pl.BlockSpec((tm, d_o), lambda i: (i, 0)),
        ],
        out_shape=[
            jax.ShapeDtypeStruct((n, d_h), jnp.float32),
            jax.ShapeDtypeStruct((n, d_o), jnp.float32),
        ],
    )(pa, pb, x, deg0, deg1, W1, b1, W2p)


def _tc_head(q0, q1, ts, deg0, deg1, b2p, tm):
    n, d = ts.shape

    def body(q0_ref, q1_ref, ts_ref, d0_ref, d1_ref, b2_ref, o_ref):
        dinv = lax.rsqrt(d0_ref[...] + d1_ref[...] + 1.0)
        z = dinv * (q0_ref[...] + q1_ref[...] + ts_ref[...]) + b2_ref[...]
        o_ref[...] = jnp.maximum(z, 0.0) + jnp.log1p(jnp.exp(-jnp.abs(z)))

    return pl.pallas_call(
        body,
        grid=(n // tm,),
        in_specs=[
            pl.BlockSpec((tm, d), lambda i: (i, 0)),
            pl.BlockSpec((tm, d), lambda i: (i, 0)),
            pl.BlockSpec((tm, d), lambda i: (i, 0)),
            pl.BlockSpec((tm, 1), lambda i: (i, 0)),
            pl.BlockSpec((tm, 1), lambda i: (i, 0)),
            pl.BlockSpec((1, d), lambda i: (0, 0)),
        ],
        out_specs=pl.BlockSpec((tm, d), lambda i: (i, 0)),
        out_shape=jax.ShapeDtypeStruct((n, d), jnp.float32),
    )(q0, q1, ts, deg0, deg1, b2p)


def kernel(x, edge_index, W1, b1, W2, b2):
    n = x.shape[0]
    e = edge_index.shape[1]

    n_pad = 8 * NS * -(-(n + DEG_W) // (8 * NS))
    cpw = -(-e // (NW * K))
    cpw += cpw % 2
    e_pad = NW * cpw * K

    row = edge_index[0].astype(jnp.int32)
    col = edge_index[1].astype(jnp.int32)
    pad = e_pad - e
    prow = jnp.zeros((pad,), jnp.int32)
    pcol = n + (jnp.arange(pad, dtype=jnp.int32) % (n_pad - n))
    rows3 = jnp.concatenate([row, prow]).reshape(NW, cpw, K)
    cols3 = jnp.concatenate([col, pcol]).reshape(NW, cpw, K)

    tm = 2000 if n % 2000 == 0 else 8 * (n // 8)

    deg = _sc_degree(cols3, n_pad, cpw)
    deg0 = deg[0, :n, :1]
    deg1 = deg[1, :n, :1]

    xs2 = _tc_scale(x, deg0, deg1, tm)
    dh = x.shape[1] // 2
    cpw2 = 2 * cpw
    rows3s = jnp.concatenate([row, prow]).reshape(NS, cpw2, K)
    cols3s = jnp.concatenate([col, pcol]).reshape(NS, cpw2, K)
    pab = _sc_scatter_sum_split(xs2, rows3s, cols3s, n_pad, dh, cpw2)

    W2p = jnp.pad(W2, ((0, 0), (0, -W2.shape[1] % DEG_W)))
    b2p = jnp.pad(b2, (0, -b2.shape[0] % DEG_W)).reshape(1, -1)
    h, ts = _tc_dense(pab[0, :n], pab[1, :n],
                      x, deg0, deg1, W1, b1.reshape(1, -1), W2p, tm)

    q = _sc_scatter_sum(ts, rows3, cols3, n_pad, W2p.shape[1], cpw)
    ev = _tc_head(q[0, :n], q[1, :n], ts, deg0, deg1, b2p, tm)
    return ev[:, : W2.shape[1]], h


# speedup vs baseline: 16.9339x; 1.1037x over previous
import functools

import jax
import jax.numpy as jnp
from jax import lax
from jax.experimental import pallas as pl
from jax.experimental.pallas import tpu as pltpu
from jax.experimental.pallas import tpu_sc as plsc

NC = 2
NS = 16
NW = NC * NS
K = 128
DEG_W = 16

_mesh = lambda: plsc.VectorSubcoreMesh(core_axis_name="c", subcore_axis_name="s")
_SC_PARAMS = pltpu.CompilerParams(use_tc_tiling_on_sc=False)


def _sc_degree(cols3, n_pad, cpw):
    rpw = n_pad // NS
    nzc, rem = divmod(rpw, K)

    @functools.partial(
        pl.kernel,
        out_type=jax.ShapeDtypeStruct((NC, n_pad, DEG_W), jnp.float32),
        mesh=_mesh(),
        compiler_params=_SC_PARAMS,
        scratch_types=[
            pltpu.VMEM((cpw, K), jnp.int32),
            pltpu.VMEM((K, DEG_W), jnp.float32),
            pltpu.VMEM((K, DEG_W), jnp.float32),
            pltpu.VMEM_SHARED((n_pad, DEG_W), jnp.float32),
        ],
    )
    def kern(cols_hbm, ones_hbm, z_hbm, out_hbm, cidx, onesb, zb, acc):
        c = lax.axis_index("c")
        s = lax.axis_index("s")
        wid = s * NC + c
        pltpu.sync_copy(cols_hbm.at[wid], cidx)
        pltpu.sync_copy(ones_hbm, onesb)
        pltpu.sync_copy(z_hbm, zb)
        base = s * rpw

        @pl.loop(0, nzc)
        def _(i):
            pltpu.sync_copy(zb, acc.at[pl.ds(base + i * K, K)])

        if rem:
            pltpu.sync_copy(zb.at[pl.ds(0, rem)], acc.at[pl.ds(base + nzc * K, rem)])
        plsc.subcore_barrier()

        @pl.loop(0, cpw)
        def _(i):
            pltpu.sync_copy(onesb, acc.at[cidx.at[i]], add=True)

        plsc.subcore_barrier()
        pltpu.sync_copy(acc.at[pl.ds(base, rpw)], out_hbm.at[c, pl.ds(base, rpw)])

    ones = jnp.ones((K, DEG_W), jnp.float32)
    zeros = jnp.zeros((K, DEG_W), jnp.float32)
    return kern(cols3, ones, zeros)


def _sc_agg_quarters(vals4, rows_s, cols_s, n_pad, dq, nch):
    rpw = n_pad // NS
    nzc, rem = divmod(rpw, K)
    assert nch % 8 == 0

    @functools.partial(
        pl.kernel,
        out_type=jax.ShapeDtypeStruct((2 * NC, n_pad, dq), jnp.float32),
        mesh=_mesh(),
        compiler_params=_SC_PARAMS,
        scratch_types=[
            pltpu.VMEM((nch, K), jnp.int32),
            pltpu.VMEM((nch, K), jnp.int32),
        ] + [pltpu.VMEM((K, dq), jnp.float32) for _ in range(8)] + [
            pltpu.VMEM_SHARED((n_pad, dq), jnp.float32),
            pltpu.SemaphoreType.DMA,
            pltpu.SemaphoreType.DMA,
        ],
    )
    def kern(vals_hbm, rows_hbm, cols_hbm, z_hbm, out_hbm,
             ridx, cidx, b0, b1, b2, b3, b4, b5, b6, b7, acc, gsem, ssem):
        c = lax.axis_index("c")
        s = lax.axis_index("s")
        bufs = (b0, b1, b2, b3, b4, b5, b6, b7)
        pltpu.sync_copy(rows_hbm.at[s], ridx)
        pltpu.sync_copy(cols_hbm.at[s], cidx)
        base = s * rpw

        for phase in range(2):
            src = vals_hbm.at[2 * c + phase]
            pltpu.sync_copy(z_hbm, b0)

            @pl.loop(0, nzc)
            def _(i):
                pltpu.sync_copy(b0, acc.at[pl.ds(base + i * K, K)])

            if rem:
                pltpu.sync_copy(b0.at[pl.ds(0, rem)],
                                acc.at[pl.ds(base + nzc * K, rem)])
            for b in range(8):
                pltpu.async_copy(src.at[ridx.at[b]], bufs[b], gsem)
            plsc.subcore_barrier()

            def do_group(group, it):
                i0 = 8 * it + 4 * group
                gbufs = bufs[4 * group:4 * group + 4]
                for b in range(4):
                    pltpu.make_async_copy(src.at[ridx.at[i0 + b]],
                                          gbufs[b], gsem).wait()
                scat = [pltpu.async_copy(gbufs[b], acc.at[cidx.at[i0 + b]],
                                         ssem, add=True) for b in range(4)]
                for dsc in scat:
                    dsc.wait()
                nxt = i0 + 8

                @pl.when(nxt < nch)
                def _():
                    for b in range(4):
                        pltpu.async_copy(src.at[ridx.at[nxt + b]],
                                         gbufs[b], gsem)

            @pl.loop(0, nch // 8)
            def _(it):
                do_group(0, it)
                do_group(1, it)

            plsc.subcore_barrier()
            pltpu.sync_copy(acc.at[pl.ds(base, rpw)],
                            out_hbm.at[2 * c + phase, pl.ds(base, rpw)])

    zeros = jnp.zeros((K, dq), jnp.float32)
    return kern(vals4, rows_s, cols_s, zeros)


def _sc_agg(vals2, rows_s, cols_s, n_pad, d, nch, by_core_half):
    rpw = n_pad // NS
    nzc, rem = divmod(rpw, K)
    assert nch % 8 == 0

    @functools.partial(
        pl.kernel,
        out_type=jax.ShapeDtypeStruct((NC, n_pad, d), jnp.float32),
        mesh=_mesh(),
        compiler_params=_SC_PARAMS,
        scratch_types=[
            pltpu.VMEM((nch, K), jnp.int32),
            pltpu.VMEM((nch, K), jnp.int32),
        ] + [pltpu.VMEM((K, d), jnp.float32) for _ in range(8)] + [
            pltpu.VMEM_SHARED((n_pad, d), jnp.float32),
            pltpu.SemaphoreType.DMA,
            pltpu.SemaphoreType.DMA,
        ],
    )
    def kern(vals_hbm, rows_hbm, cols_hbm, z_hbm, out_hbm,
             ridx, cidx, b0, b1, b2, b3, b4, b5, b6, b7, acc, gsem, ssem):
        c = lax.axis_index("c")
        s = lax.axis_index("s")
        slab = s if by_core_half else s * NC + c
        src = vals_hbm.at[c]
        bufs = (b0, b1, b2, b3, b4, b5, b6, b7)
        pltpu.sync_copy(rows_hbm.at[slab], ridx)
        pltpu.sync_copy(cols_hbm.at[slab], cidx)
        pltpu.sync_copy(z_hbm, b0)

        base = s * rpw

        @pl.loop(0, nzc)
        def _(i):
            pltpu.sync_copy(b0, acc.at[pl.ds(base + i * K, K)])

        if rem:
            pltpu.sync_copy(b0.at[pl.ds(0, rem)], acc.at[pl.ds(base + nzc * K, rem)])
        for b in range(8):
            pltpu.async_copy(src.at[ridx.at[b]], bufs[b], gsem)
        plsc.subcore_barrier()

        def do_group(group, it):
            i0 = 8 * it + 4 * group
            gbufs = bufs[4 * group:4 * group + 4]
            for b in range(4):
                pltpu.make_async_copy(src.at[ridx.at[i0 + b]], gbufs[b], gsem).wait()
            scat = [pltpu.async_copy(gbufs[b], acc.at[cidx.at[i0 + b]], ssem,
                                     add=True) for b in range(4)]
            for dsc in scat:
                dsc.wait()
            nxt = i0 + 8

            @pl.when(nxt < nch)
            def _():
                for b in range(4):
                    pltpu.async_copy(src.at[ridx.at[nxt + b]], gbufs[b], gsem)

        @pl.loop(0, nch // 8)
        def _(it):
            do_group(0, it)
            do_group(1, it)

        plsc.subcore_barrier()
        pltpu.sync_copy(acc.at[pl.ds(base, rpw)], out_hbm.at[c, pl.ds(base, rpw)])

    zeros = jnp.zeros((K, d), jnp.float32)
    return kern(vals2, rows_s, cols_s, zeros)


def _tc_scale(x, deg0, deg1, tm):
    n, d = x.shape
    dq = d // 4

    def body(x_ref, d0_ref, d1_ref, o_ref):
        dinv = lax.rsqrt(d0_ref[...] + d1_ref[...] + 1.0)
        xs = dinv * x_ref[...]
        for q in range(4):
            o_ref[q] = xs[:, q * dq:(q + 1) * dq]

    return pl.pallas_call(
        body,
        grid=(n // tm,),
        in_specs=[
            pl.BlockSpec((tm, d), lambda i: (i, 0)),
            pl.BlockSpec((tm, 1), lambda i: (i, 0)),
            pl.BlockSpec((tm, 1), lambda i: (i, 0)),
        ],
        out_specs=pl.BlockSpec((4, tm, dq), lambda i: (0, i, 0)),
        out_shape=jax.ShapeDtypeStruct((4, n, dq), jnp.float32),
    )(x, deg0, deg1)


def _tc_dense(pq, x, deg0, deg1, W1, b1, W2p, tm):
    n, d_in = x.shape
    dq = d_in // 4
    d_h = W1.shape[1]
    d_o = W2p.shape[1]

    def body(pq_ref, x_ref, d0_ref, d1_ref,
             w1_ref, b1_ref, w2_ref, h_ref, ts_ref):
        dinv = lax.rsqrt(d0_ref[...] + d1_ref[...] + 1.0)
        d2 = dinv * dinv
        x_blk = x_ref[...]
        w1 = w1_ref[...]
        h = b1_ref[...]
        for q in range(4):
            agg_q = (dinv * pq_ref[q]
                     + d2 * x_blk[:, q * dq:(q + 1) * dq])
            h = h + jnp.dot(agg_q, w1[q * dq:(q + 1) * dq],
                            preferred_element_type=jnp.float32)
        h = jnp.maximum(h, 0.0)
        h_ref[...] = h
        t = jnp.dot(h, w2_ref[...], preferred_element_type=jnp.float32)
        tsv = dinv * t
        ts_ref[0] = tsv
        ts_ref[1] = tsv

    return pl.pallas_call(
        body,
        grid=(n // tm,),
        in_specs=[
            pl.BlockSpec((4, tm, dq), lambda i: (0, i, 0)),
            pl.BlockSpec((tm, d_in), lambda i: (i, 0)),
            pl.BlockSpec((tm, 1), lambda i: (i, 0)),
            pl.BlockSpec((tm, 1), lambda i: (i, 0)),
            pl.BlockSpec((d_in, d_h), lambda i: (0, 0)),
            pl.BlockSpec((1, d_h), lambda i: (0, 0)),
            pl.BlockSpec((d_h, d_o), lambda i: (0, 0)),
        ],
        out_specs=[
            pl.BlockSpec((tm, d_h), lambda i: (i, 0)),
            pl.BlockSpec((2, tm, d_o), lambda i: (0, i, 0)),
        ],
        out_shape=[
            jax.ShapeDtypeStruct((n, d_h), jnp.float32),
            jax.ShapeDtypeStruct((2, n, d_o), jnp.float32),
        ],
    )(pq, x, deg0, deg1, W1, b1, W2p)


def _tc_head(q0, q1, ts, deg0, deg1, b2p, tm):
    n, d = ts.shape

    def body(q0_ref, q1_ref, ts_ref, d0_ref, d1_ref, b2_ref, o_ref):
        dinv = lax.rsqrt(d0_ref[...] + d1_ref[...] + 1.0)
        z = dinv * (q0_ref[...] + q1_ref[...] + ts_ref[...]) + b2_ref[...]
        o_ref[...] = jnp.maximum(z, 0.0) + jnp.log1p(jnp.exp(-jnp.abs(z)))

    return pl.pallas_call(
        body,
        grid=(n // tm,),
        in_specs=[
            pl.BlockSpec((tm, d), lambda i: (i, 0)),
            pl.BlockSpec((tm, d), lambda i: (i, 0)),
            pl.BlockSpec((tm, d), lambda i: (i, 0)),
            pl.BlockSpec((tm, 1), lambda i: (i, 0)),
            pl.BlockSpec((tm, 1), lambda i: (i, 0)),
            pl.BlockSpec((1, d), lambda i: (0, 0)),
        ],
        out_specs=pl.BlockSpec((tm, d), lambda i: (i, 0)),
        out_shape=jax.ShapeDtypeStruct((n, d), jnp.float32),
    )(q0, q1, ts, deg0, deg1, b2p)


def kernel(x, edge_index, W1, b1, W2, b2):
    n = x.shape[0]
    e = edge_index.shape[1]

    n_pad = 8 * NS * -(-(n + DEG_W) // (8 * NS))
    cpw = 8 * -(-e // (NW * K * 8))
    e_pad = NW * cpw * K

    row = edge_index[0].astype(jnp.int32)
    col = edge_index[1].astype(jnp.int32)
    pad = e_pad - e
    prow = jnp.zeros((pad,), jnp.int32)
    pcol = n + (jnp.arange(pad, dtype=jnp.int32) % (n_pad - n))
    rows3 = jnp.concatenate([row, prow]).reshape(NW, cpw, K)
    cols3 = jnp.concatenate([col, pcol]).reshape(NW, cpw, K)

    tm = 2000 if n % 2000 == 0 else 8 * (n // 8)

    deg = _sc_degree(cols3, n_pad, cpw)
    deg0 = deg[0, :n, :1]
    deg1 = deg[1, :n, :1]

    xs4 = _tc_scale(x, deg0, deg1, tm)
    dq = x.shape[1] // 4
    cpw2 = 2 * cpw
    rows3s = jnp.concatenate([row, prow]).reshape(NS, cpw2, K)
    cols3s = jnp.concatenate([col, pcol]).reshape(NS, cpw2, K)
    pquart = _sc_agg_quarters(xs4, rows3s, cols3s, n_pad, dq, cpw2)

    W2p = jnp.pad(W2, ((0, 0), (0, -W2.shape[1] % DEG_W)))
    b2p = jnp.pad(b2, (0, -b2.shape[0] % DEG_W)).reshape(1, -1)
    h, ts2 = _tc_dense(pquart[:, :n], x, deg0, deg1,
                       b1=b1.reshape(1, -1), W1=W1, W2p=W2p, tm=tm)

    q = _sc_agg(ts2, rows3, cols3, n_pad, W2p.shape[1], cpw, by_core_half=False)
    ev = _tc_head(q[0, :n], q[1, :n], ts2[0], deg0, deg1, b2p, tm)
    return ev[:, : W2.shape[1]], h


# speedup vs baseline: 17.3828x; 1.0265x over previous
import functools

import jax
import jax.numpy as jnp
from jax import lax
from jax.experimental import pallas as pl
from jax.experimental.pallas import tpu as pltpu
from jax.experimental.pallas import tpu_sc as plsc

NC = 2
NS = 16
NW = NC * NS
K = 128
DEG_W = 16

_mesh = lambda: plsc.VectorSubcoreMesh(core_axis_name="c", subcore_axis_name="s")
_SC_PARAMS = pltpu.CompilerParams(use_tc_tiling_on_sc=False)


def _sc_degree(cols3, n_pad, cpw):
    rpw = n_pad // NS
    nzc, rem = divmod(rpw, K)

    @functools.partial(
        pl.kernel,
        out_type=jax.ShapeDtypeStruct((NC, n_pad, DEG_W), jnp.float32),
        mesh=_mesh(),
        compiler_params=_SC_PARAMS,
        scratch_types=[
            pltpu.VMEM((cpw, K), jnp.int32),
            pltpu.VMEM((K, DEG_W), jnp.float32),
            pltpu.VMEM((K, DEG_W), jnp.float32),
            pltpu.VMEM_SHARED((n_pad, DEG_W), jnp.float32),
        ],
    )
    def kern(cols_hbm, ones_hbm, z_hbm, out_hbm, cidx, onesb, zb, acc):
        c = lax.axis_index("c")
        s = lax.axis_index("s")
        wid = s * NC + c
        pltpu.sync_copy(cols_hbm.at[wid], cidx)
        pltpu.sync_copy(ones_hbm, onesb)
        pltpu.sync_copy(z_hbm, zb)
        base = s * rpw

        @pl.loop(0, nzc)
        def _(i):
            pltpu.sync_copy(zb, acc.at[pl.ds(base + i * K, K)])

        if rem:
            pltpu.sync_copy(zb.at[pl.ds(0, rem)], acc.at[pl.ds(base + nzc * K, rem)])
        plsc.subcore_barrier()

        @pl.loop(0, cpw)
        def _(i):
            pltpu.sync_copy(onesb, acc.at[cidx.at[i]], add=True)

        plsc.subcore_barrier()
        pltpu.sync_copy(acc.at[pl.ds(base, rpw)], out_hbm.at[c, pl.ds(base, rpw)])

    ones = jnp.ones((K, DEG_W), jnp.float32)
    zeros = jnp.zeros((K, DEG_W), jnp.float32)
    return kern(cols3, ones, zeros)


def _sc_agg_quarters(vals4, rows_s, cols_s, n_pad, dq, nch):
    rpw = n_pad // NS
    nzc, rem = divmod(rpw, K)
    assert nch % 8 == 0

    @functools.partial(
        pl.kernel,
        out_type=jax.ShapeDtypeStruct((2 * NC, n_pad, dq), jnp.float32),
        mesh=_mesh(),
        compiler_params=_SC_PARAMS,
        scratch_types=[
            pltpu.VMEM((nch, K), jnp.int32),
            pltpu.VMEM((nch, K), jnp.int32),
        ] + [pltpu.VMEM((K, dq), jnp.float32) for _ in range(8)] + [
            pltpu.VMEM_SHARED((n_pad, dq), jnp.float32),
            pltpu.SemaphoreType.DMA,
            pltpu.SemaphoreType.DMA,
        ],
    )
    def kern(vals_hbm, rows_hbm, cols_hbm, z_hbm, out_hbm,
             ridx, cidx, b0, b1, b2, b3, b4, b5, b6, b7, acc, gsem, ssem):
        c = lax.axis_index("c")
        s = lax.axis_index("s")
        bufs = (b0, b1, b2, b3, b4, b5, b6, b7)
        pltpu.sync_copy(rows_hbm.at[s], ridx)
        pltpu.sync_copy(cols_hbm.at[s], cidx)
        base = s * rpw

        for phase in range(2):
            src = vals_hbm.at[2 * c + phase]
            pltpu.sync_copy(z_hbm, b0)

            @pl.loop(0, nzc)
            def _(i):
                pltpu.sync_copy(b0, acc.at[pl.ds(base + i * K, K)])

            if rem:
                pltpu.sync_copy(b0.at[pl.ds(0, rem)],
                                acc.at[pl.ds(base + nzc * K, rem)])
            for b in range(8):
                pltpu.async_copy(src.at[ridx.at[b]], bufs[b], gsem)
            plsc.subcore_barrier()

            def do_group(group, it):
                i0 = 8 * it + 4 * group
                gbufs = bufs[4 * group:4 * group + 4]
                for b in range(4):
                    pltpu.make_async_copy(src.at[ridx.at[i0 + b]],
                                          gbufs[b], gsem).wait()
                scat = [pltpu.async_copy(gbufs[b], acc.at[cidx.at[i0 + b]],
                                         ssem, add=True) for b in range(4)]
                for dsc in scat:
                    dsc.wait()
                nxt = i0 + 8

                @pl.when(nxt < nch)
                def _():
                    for b in range(4):
                        pltpu.async_copy(src.at[ridx.at[nxt + b]],
                                         gbufs[b], gsem)

            @pl.loop(0, nch // 8)
            def _(it):
                do_group(0, it)
                do_group(1, it)

            plsc.subcore_barrier()
            pltpu.sync_copy(acc.at[pl.ds(base, rpw)],
                            out_hbm.at[2 * c + phase, pl.ds(base, rpw)])

    zeros = jnp.zeros((K, dq), jnp.float32)
    return kern(vals4, rows_s, cols_s, zeros)


def _sc_agg(vals2, rows_s, cols_s, n_pad, d, nch, by_core_half):
    rpw = n_pad // NS
    nzc, rem = divmod(rpw, K)
    assert nch % 8 == 0

    @functools.partial(
        pl.kernel,
        out_type=jax.ShapeDtypeStruct((NC, n_pad, d), jnp.float32),
        mesh=_mesh(),
        compiler_params=_SC_PARAMS,
        scratch_types=[
            pltpu.VMEM((nch, K), jnp.int32),
            pltpu.VMEM((nch, K), jnp.int32),
        ] + [pltpu.VMEM((K, d), jnp.float32) for _ in range(8)] + [
            pltpu.VMEM_SHARED((n_pad, d), jnp.float32),
            pltpu.SemaphoreType.DMA,
            pltpu.SemaphoreType.DMA,
        ],
    )
    def kern(vals_hbm, rows_hbm, cols_hbm, z_hbm, out_hbm,
             ridx, cidx, b0, b1, b2, b3, b4, b5, b6, b7, acc, gsem, ssem):
        c = lax.axis_index("c")
        s = lax.axis_index("s")
        slab = s if by_core_half else s * NC + c
        src = vals_hbm.at[c]
        bufs = (b0, b1, b2, b3, b4, b5, b6, b7)
        pltpu.sync_copy(rows_hbm.at[slab], ridx)
        pltpu.sync_copy(cols_hbm.at[slab], cidx)
        pltpu.sync_copy(z_hbm, b0)

        base = s * rpw

        @pl.loop(0, nzc)
        def _(i):
            pltpu.sync_copy(b0, acc.at[pl.ds(base + i * K, K)])

        if rem:
            pltpu.sync_copy(b0.at[pl.ds(0, rem)], acc.at[pl.ds(base + nzc * K, rem)])
        for b in range(8):
            pltpu.async_copy(src.at[ridx.at[b]], bufs[b], gsem)
        plsc.subcore_barrier()

        def do_group(group, it):
            i0 = 8 * it + 4 * group
            gbufs = bufs[4 * group:4 * group + 4]
            for b in range(4):
                pltpu.make_async_copy(src.at[ridx.at[i0 + b]], gbufs[b], gsem).wait()
            scat = [pltpu.async_copy(gbufs[b], acc.at[cidx.at[i0 + b]], ssem,
                                     add=True) for b in range(4)]
            for dsc in scat:
                dsc.wait()
            nxt = i0 + 8

            @pl.when(nxt < nch)
            def _():
                for b in range(4):
                    pltpu.async_copy(src.at[ridx.at[nxt + b]], gbufs[b], gsem)

        @pl.loop(0, nch // 8)
        def _(it):
            do_group(0, it)
            do_group(1, it)

        plsc.subcore_barrier()
        pltpu.sync_copy(acc.at[pl.ds(base, rpw)], out_hbm.at[c, pl.ds(base, rpw)])

    zeros = jnp.zeros((K, d), jnp.float32)
    return kern(vals2, rows_s, cols_s, zeros)


def _tc_scale(x, deg0, deg1, tm):
    n, d = x.shape
    dq = d // 4

    def body(x_ref, d0_ref, d1_ref, o_ref):
        dinv = lax.rsqrt(d0_ref[...] + d1_ref[...] + 1.0)
        xs = dinv * x_ref[...]
        for q in range(4):
            o_ref[q] = xs[:, q * dq:(q + 1) * dq]

    return pl.pallas_call(
        body,
        grid=(n // tm,),
        in_specs=[
            pl.BlockSpec((tm, d), lambda i: (i, 0)),
            pl.BlockSpec((tm, 1), lambda i: (i, 0)),
            pl.BlockSpec((tm, 1), lambda i: (i, 0)),
        ],
        out_specs=pl.BlockSpec((4, tm, dq), lambda i: (0, i, 0)),
        out_shape=jax.ShapeDtypeStruct((4, n, dq), jnp.float32),
    )(x, deg0, deg1)


def _tc_dense(pq, x, deg0, deg1, W1, b1, W2p, tm):
    n, d_in = x.shape
    dq = d_in // 4
    d_h = W1.shape[1]
    d_o = W2p.shape[1]

    def body(pq_ref, x_ref, d0_ref, d1_ref,
             w1_ref, b1_ref, w2_ref, h_ref, ts_ref):
        dinv = lax.rsqrt(d0_ref[...] + d1_ref[...] + 1.0)
        d2 = dinv * dinv
        x_blk = x_ref[...]
        w1 = w1_ref[...]
        h = b1_ref[...]
        for q in range(4):
            agg_q = (dinv * pq_ref[q]
                     + d2 * x_blk[:, q * dq:(q + 1) * dq])
            h = h + jnp.dot(agg_q, w1[q * dq:(q + 1) * dq],
                            preferred_element_type=jnp.float32)
        h = jnp.maximum(h, 0.0)
        h_ref[...] = h
        t = jnp.dot(h, w2_ref[...], preferred_element_type=jnp.float32)
        tsv = dinv * t
        ts_ref[0] = tsv[:, :d_o // 2]
        ts_ref[1] = tsv[:, d_o // 2:]

    return pl.pallas_call(
        body,
        grid=(n // tm,),
        in_specs=[
            pl.BlockSpec((4, tm, dq), lambda i: (0, i, 0)),
            pl.BlockSpec((tm, d_in), lambda i: (i, 0)),
            pl.BlockSpec((tm, 1), lambda i: (i, 0)),
            pl.BlockSpec((tm, 1), lambda i: (i, 0)),
            pl.BlockSpec((d_in, d_h), lambda i: (0, 0)),
            pl.BlockSpec((1, d_h), lambda i: (0, 0)),
            pl.BlockSpec((d_h, d_o), lambda i: (0, 0)),
        ],
        out_specs=[
            pl.BlockSpec((tm, d_h), lambda i: (i, 0)),
            pl.BlockSpec((2, tm, d_o // 2), lambda i: (0, i, 0)),
        ],
        out_shape=[
            jax.ShapeDtypeStruct((n, d_h), jnp.float32),
            jax.ShapeDtypeStruct((2, n, d_o // 2), jnp.float32),
        ],
    )(pq, x, deg0, deg1, W1, b1, W2p)


def _tc_head(q2, ts2, deg0, deg1, b2p, tm):
    n = ts2.shape[1]
    dh = ts2.shape[2]

    def body(q_ref, ts_ref, d0_ref, d1_ref, b2_ref, o_ref):
        dinv = lax.rsqrt(d0_ref[...] + d1_ref[...] + 1.0)
        for half in range(2):
            z = (dinv * (q_ref[half] + ts_ref[half])
                 + b2_ref[:, half * dh:(half + 1) * dh])
            o_ref[:, half * dh:(half + 1) * dh] = (
                jnp.maximum(z, 0.0) + jnp.log1p(jnp.exp(-jnp.abs(z))))

    return pl.pallas_call(
        body,
        grid=(n // tm,),
        in_specs=[
            pl.BlockSpec((2, tm, dh), lambda i: (0, i, 0)),
            pl.BlockSpec((2, tm, dh), lambda i: (0, i, 0)),
            pl.BlockSpec((tm, 1), lambda i: (i, 0)),
            pl.BlockSpec((tm, 1), lambda i: (i, 0)),
            pl.BlockSpec((1, 2 * dh), lambda i: (0, 0)),
        ],
        out_specs=pl.BlockSpec((tm, 2 * dh), lambda i: (i, 0)),
        out_shape=jax.ShapeDtypeStruct((n, 2 * dh), jnp.float32),
    )(q2, ts2, deg0, deg1, b2p)


def kernel(x, edge_index, W1, b1, W2, b2):
    n = x.shape[0]
    e = edge_index.shape[1]

    n_pad = 8 * NS * -(-(n + DEG_W) // (8 * NS))
    cpw = 8 * -(-e // (NW * K * 8))
    e_pad = NW * cpw * K

    row = edge_index[0].astype(jnp.int32)
    col = edge_index[1].astype(jnp.int32)
    pad = e_pad - e
    prow = jnp.zeros((pad,), jnp.int32)
    pcol = n + (jnp.arange(pad, dtype=jnp.int32) % (n_pad - n))
    rows3 = jnp.concatenate([row, prow]).reshape(NW, cpw, K)
    cols3 = jnp.concatenate([col, pcol]).reshape(NW, cpw, K)

    tm = 2000 if n % 2000 == 0 else 8 * (n // 8)

    deg = _sc_degree(cols3, n_pad, cpw)
    deg0 = deg[0, :n, :1]
    deg1 = deg[1, :n, :1]

    xs4 = _tc_scale(x, deg0, deg1, tm)
    dq = x.shape[1] // 4
    cpw2 = 2 * cpw
    rows3s = jnp.concatenate([row, prow]).reshape(NS, cpw2, K)
    cols3s = jnp.concatenate([col, pcol]).reshape(NS, cpw2, K)
    pquart = _sc_agg_quarters(xs4, rows3s, cols3s, n_pad, dq, cpw2)

    wpad = -W2.shape[1] % (2 * dq)
    W2p = jnp.pad(W2, ((0, 0), (0, wpad)))
    b2p = jnp.pad(b2, (0, wpad)).reshape(1, -1)
    h, ts2 = _tc_dense(pquart[:, :n], x, deg0, deg1,
                       b1=b1.reshape(1, -1), W1=W1, W2p=W2p, tm=tm)

    q2 = _sc_agg(ts2, rows3s, cols3s, n_pad, W2p.shape[1] // 2, cpw2,
                 by_core_half=True)
    ev = _tc_head(q2[:, :n], ts2, deg0, deg1, b2p, tm)
    return ev[:, : W2.shape[1]], h


# speedup vs baseline: 17.6003x; 1.0125x over previous
import functools

import jax
import jax.numpy as jnp
from jax import lax
from jax.experimental import pallas as pl
from jax.experimental.pallas import tpu as pltpu
from jax.experimental.pallas import tpu_sc as plsc

NC = 2
NS = 16
NW = NC * NS
K = 256
DEG_W = 16

_mesh = lambda: plsc.VectorSubcoreMesh(core_axis_name="c", subcore_axis_name="s")
_SC_PARAMS = pltpu.CompilerParams(use_tc_tiling_on_sc=False)


def _sc_degree(cols3, n_pad, cpw):
    rpw = n_pad // NS
    nzc, rem = divmod(rpw, K)

    @functools.partial(
        pl.kernel,
        out_type=jax.ShapeDtypeStruct((NC, n_pad, DEG_W), jnp.float32),
        mesh=_mesh(),
        compiler_params=_SC_PARAMS,
        scratch_types=[
            pltpu.VMEM((cpw, K), jnp.int32),
            pltpu.VMEM((K, DEG_W), jnp.float32),
            pltpu.VMEM((K, DEG_W), jnp.float32),
            pltpu.VMEM_SHARED((n_pad, DEG_W), jnp.float32),
        ],
    )
    def kern(cols_hbm, ones_hbm, z_hbm, out_hbm, cidx, onesb, zb, acc):
        c = lax.axis_index("c")
        s = lax.axis_index("s")
        wid = s * NC + c
        pltpu.sync_copy(cols_hbm.at[wid], cidx)
        pltpu.sync_copy(ones_hbm, onesb)
        pltpu.sync_copy(z_hbm, zb)
        base = s * rpw

        @pl.loop(0, nzc)
        def _(i):
            pltpu.sync_copy(zb, acc.at[pl.ds(base + i * K, K)])

        if rem:
            pltpu.sync_copy(zb.at[pl.ds(0, rem)], acc.at[pl.ds(base + nzc * K, rem)])
        plsc.subcore_barrier()

        @pl.loop(0, cpw)
        def _(i):
            pltpu.sync_copy(onesb, acc.at[cidx.at[i]], add=True)

        plsc.subcore_barrier()
        pltpu.sync_copy(acc.at[pl.ds(base, rpw)], out_hbm.at[c, pl.ds(base, rpw)])

    ones = jnp.ones((K, DEG_W), jnp.float32)
    zeros = jnp.zeros((K, DEG_W), jnp.float32)
    return kern(cols3, ones, zeros)


def _sc_agg_quarters(vals4, rows_s, cols_s, n_pad, dq, nch):
    rpw = n_pad // NS
    nzc, rem = divmod(rpw, K)
    assert nch % 8 == 0

    @functools.partial(
        pl.kernel,
        out_type=jax.ShapeDtypeStruct((2 * NC, n_pad, dq), jnp.float32),
        mesh=_mesh(),
        compiler_params=_SC_PARAMS,
        scratch_types=[
            pltpu.VMEM((nch, K), jnp.int32),
            pltpu.VMEM((nch, K), jnp.int32),
        ] + [pltpu.VMEM((K, dq), jnp.float32) for _ in range(8)] + [
            pltpu.VMEM_SHARED((n_pad, dq), jnp.float32),
            pltpu.SemaphoreType.DMA,
            pltpu.SemaphoreType.DMA,
        ],
    )
    def kern(vals_hbm, rows_hbm, cols_hbm, z_hbm, out_hbm,
             ridx, cidx, b0, b1, b2, b3, b4, b5, b6, b7, acc, gsem, ssem):
        c = lax.axis_index("c")
        s = lax.axis_index("s")
        bufs = (b0, b1, b2, b3, b4, b5, b6, b7)
        pltpu.sync_copy(rows_hbm.at[s], ridx)
        pltpu.sync_copy(cols_hbm.at[s], cidx)
        base = s * rpw

        for phase in range(2):
            src = vals_hbm.at[2 * c + phase]
            pltpu.sync_copy(z_hbm, b0)

            @pl.loop(0, nzc)
            def _(i):
                pltpu.sync_copy(b0, acc.at[pl.ds(base + i * K, K)])

            if rem:
                pltpu.sync_copy(b0.at[pl.ds(0, rem)],
                                acc.at[pl.ds(base + nzc * K, rem)])
            for b in range(8):
                pltpu.async_copy(src.at[ridx.at[b]], bufs[b], gsem)
            plsc.subcore_barrier()

            def do_group(group, it):
                i0 = 8 * it + 4 * group
                gbufs = bufs[4 * group:4 * group + 4]
                for b in range(4):
                    pltpu.make_async_copy(src.at[ridx.at[i0 + b]],
                                          gbufs[b], gsem).wait()
                scat = [pltpu.async_copy(gbufs[b], acc.at[cidx.at[i0 + b]],
                                         ssem, add=True) for b in range(4)]
                for dsc in scat:
                    dsc.wait()
                nxt = i0 + 8

                @pl.when(nxt < nch)
                def _():
                    for b in range(4):
                        pltpu.async_copy(src.at[ridx.at[nxt + b]],
                                         gbufs[b], gsem)

            @pl.loop(0, nch // 8)
            def _(it):
                do_group(0, it)
                do_group(1, it)

            plsc.subcore_barrier()
            pltpu.sync_copy(acc.at[pl.ds(base, rpw)],
                            out_hbm.at[2 * c + phase, pl.ds(base, rpw)])

    zeros = jnp.zeros((K, dq), jnp.float32)
    return kern(vals4, rows_s, cols_s, zeros)


def _sc_agg(vals2, rows_s, cols_s, n_pad, d, nch, by_core_half):
    rpw = n_pad // NS
    nzc, rem = divmod(rpw, K)
    assert nch % 8 == 0

    @functools.partial(
        pl.kernel,
        out_type=jax.ShapeDtypeStruct((NC, n_pad, d), jnp.float32),
        mesh=_mesh(),
        compiler_params=_SC_PARAMS,
        scratch_types=[
            pltpu.VMEM((nch, K), jnp.int32),
            pltpu.VMEM((nch, K), jnp.int32),
        ] + [pltpu.VMEM((K, d), jnp.float32) for _ in range(8)] + [
            pltpu.VMEM_SHARED((n_pad, d), jnp.float32),
            pltpu.SemaphoreType.DMA,
            pltpu.SemaphoreType.DMA,
        ],
    )
    def kern(vals_hbm, rows_hbm, cols_hbm, z_hbm, out_hbm,
             ridx, cidx, b0, b1, b2, b3, b4, b5, b6, b7, acc, gsem, ssem):
        c = lax.axis_index("c")
        s = lax.axis_index("s")
        slab = s if by_core_half else s * NC + c
        src = vals_hbm.at[c]
        bufs = (b0, b1, b2, b3, b4, b5, b6, b7)
        pltpu.sync_copy(rows_hbm.at[slab], ridx)
        pltpu.sync_copy(cols_hbm.at[slab], cidx)
        pltpu.sync_copy(z_hbm, b0)

        base = s * rpw

        @pl.loop(0, nzc)
        def _(i):
            pltpu.sync_copy(b0, acc.at[pl.ds(base + i * K, K)])

        if rem:
            pltpu.sync_copy(b0.at[pl.ds(0, rem)], acc.at[pl.ds(base + nzc * K, rem)])
        for b in range(8):
            pltpu.async_copy(src.at[ridx.at[b]], bufs[b], gsem)
        plsc.subcore_barrier()

        def do_group(group, it):
            i0 = 8 * it + 4 * group
            gbufs = bufs[4 * group:4 * group + 4]
            for b in range(4):
                pltpu.make_async_copy(src.at[ridx.at[i0 + b]], gbufs[b], gsem).wait()
            scat = [pltpu.async_copy(gbufs[b], acc.at[cidx.at[i0 + b]], ssem,
                                     add=True) for b in range(4)]
            for dsc in scat:
                dsc.wait()
            nxt = i0 + 8

            @pl.when(nxt < nch)
            def _():
                for b in range(4):
                    pltpu.async_copy(src.at[ridx.at[nxt + b]], gbufs[b], gsem)

        @pl.loop(0, nch // 8)
        def _(it):
            do_group(0, it)
            do_group(1, it)

        plsc.subcore_barrier()
        pltpu.sync_copy(acc.at[pl.ds(base, rpw)], out_hbm.at[c, pl.ds(base, rpw)])

    zeros = jnp.zeros((K, d), jnp.float32)
    return kern(vals2, rows_s, cols_s, zeros)


def _tc_scale(x, deg0, deg1, tm):
    n, d = x.shape
    dq = d // 4

    def body(x_ref, d0_ref, d1_ref, o_ref):
        dinv = lax.rsqrt(d0_ref[...] + d1_ref[...] + 1.0)
        xs = dinv * x_ref[...]
        for q in range(4):
            o_ref[q] = xs[:, q * dq:(q + 1) * dq]

    return pl.pallas_call(
        body,
        grid=(n // tm,),
        in_specs=[
            pl.BlockSpec((tm, d), lambda i: (i, 0)),
            pl.BlockSpec((tm, 1), lambda i: (i, 0)),
            pl.BlockSpec((tm, 1), lambda i: (i, 0)),
        ],
        out_specs=pl.BlockSpec((4, tm, dq), lambda i: (0, i, 0)),
        out_shape=jax.ShapeDtypeStruct((4, n, dq), jnp.float32),
    )(x, deg0, deg1)


def _tc_dense(pq, x, deg0, deg1, W1, b1, W2p, tm):
    n, d_in = x.shape
    dq = d_in // 4
    d_h = W1.shape[1]
    d_o = W2p.shape[1]

    def body(pq_ref, x_ref, d0_ref, d1_ref,
             w1_ref, b1_ref, w2_ref, h_ref, ts_ref):
        dinv = lax.rsqrt(d0_ref[...] + d1_ref[...] + 1.0)
        d2 = dinv * dinv
        x_blk = x_ref[...]
        w1 = w1_ref[...]
        h = b1_ref[...]
        for q in range(4):
            agg_q = (dinv * pq_ref[q]
                     + d2 * x_blk[:, q * dq:(q + 1) * dq])
            h = h + jnp.dot(agg_q, w1[q * dq:(q + 1) * dq],
                            preferred_element_type=jnp.float32)
        h = jnp.maximum(h, 0.0)
        h_ref[...] = h
        t = jnp.dot(h, w2_ref[...], preferred_element_type=jnp.float32)
        tsv = dinv * t
        ts_ref[0] = tsv[:, :d_o // 2]
        ts_ref[1] = tsv[:, d_o // 2:]

    return pl.pallas_call(
        body,
        grid=(n // tm,),
        in_specs=[
            pl.BlockSpec((4, tm, dq), lambda i: (0, i, 0)),
            pl.BlockSpec((tm, d_in), lambda i: (i, 0)),
            pl.BlockSpec((tm, 1), lambda i: (i, 0)),
            pl.BlockSpec((tm, 1), lambda i: (i, 0)),
            pl.BlockSpec((d_in, d_h), lambda i: (0, 0)),
            pl.BlockSpec((1, d_h), lambda i: (0, 0)),
            pl.BlockSpec((d_h, d_o), lambda i: (0, 0)),
        ],
        out_specs=[
            pl.BlockSpec((tm, d_h), lambda i: (i, 0)),
            pl.BlockSpec((2, tm, d_o // 2), lambda i: (0, i, 0)),
        ],
        out_shape=[
            jax.ShapeDtypeStruct((n, d_h), jnp.float32),
            jax.ShapeDtypeStruct((2, n, d_o // 2), jnp.float32),
        ],
    )(pq, x, deg0, deg1, W1, b1, W2p)


def _tc_head(q2, ts2, deg0, deg1, b2p, tm):
    n = ts2.shape[1]
    dh = ts2.shape[2]

    def body(q_ref, ts_ref, d0_ref, d1_ref, b2_ref, o_ref):
        dinv = lax.rsqrt(d0_ref[...] + d1_ref[...] + 1.0)
        for half in range(2):
            z = (dinv * (q_ref[half] + ts_ref[half])
                 + b2_ref[:, half * dh:(half + 1) * dh])
            o_ref[:, half * dh:(half + 1) * dh] = (
                jnp.maximum(z, 0.0) + jnp.log1p(jnp.exp(-jnp.abs(z))))

    return pl.pallas_call(
        body,
        grid=(n // tm,),
        in_specs=[
            pl.BlockSpec((2, tm, dh), lambda i: (0, i, 0)),
            pl.BlockSpec((2, tm, dh), lambda i: (0, i, 0)),
            pl.BlockSpec((tm, 1), lambda i: (i, 0)),
            pl.BlockSpec((tm, 1), lambda i: (i, 0)),
            pl.BlockSpec((1, 2 * dh), lambda i: (0, 0)),
        ],
        out_specs=pl.BlockSpec((tm, 2 * dh), lambda i: (i, 0)),
        out_shape=jax.ShapeDtypeStruct((n, 2 * dh), jnp.float32),
    )(q2, ts2, deg0, deg1, b2p)


def kernel(x, edge_index, W1, b1, W2, b2):
    n = x.shape[0]
    e = edge_index.shape[1]

    n_pad = 8 * NS * -(-(n + DEG_W) // (8 * NS))
    cpw = 8 * -(-e // (NW * K * 8))
    e_pad = NW * cpw * K

    row = edge_index[0].astype(jnp.int32)
    col = edge_index[1].astype(jnp.int32)
    pad = e_pad - e
    prow = jnp.zeros((pad,), jnp.int32)
    pcol = n + (jnp.arange(pad, dtype=jnp.int32) % (n_pad - n))
    rows3 = jnp.concatenate([row, prow]).reshape(NW, cpw, K)
    cols3 = jnp.concatenate([col, pcol]).reshape(NW, cpw, K)

    tm = 2000 if n % 2000 == 0 else 8 * (n // 8)

    deg = _sc_degree(cols3, n_pad, cpw)
    deg0 = deg[0, :n, :1]
    deg1 = deg[1, :n, :1]

    xs4 = _tc_scale(x, deg0, deg1, tm)
    dq = x.shape[1] // 4
    cpw2 = 2 * cpw
    rows3s = jnp.concatenate([row, prow]).reshape(NS, cpw2, K)
    cols3s = jnp.concatenate([col, pcol]).reshape(NS, cpw2, K)
    pquart = _sc_agg_quarters(xs4, rows3s, cols3s, n_pad, dq, cpw2)

    wpad = -W2.shape[1] % (2 * dq)
    W2p = jnp.pad(W2, ((0, 0), (0, wpad)))
    b2p = jnp.pad(b2, (0, wpad)).reshape(1, -1)
    h, ts2 = _tc_dense(pquart[:, :n], x, deg0, deg1,
                       b1=b1.reshape(1, -1), W1=W1, W2p=W2p, tm=tm)

    q2 = _sc_agg(ts2, rows3s, cols3s, n_pad, W2p.shape[1] // 2, cpw2,
                 by_core_half=True)
    ev = _tc_head(q2[:, :n], ts2, deg0, deg1, b2p, tm)
    return ev[:, : W2.shape[1]], h


# speedup vs baseline: 20.3987x; 1.1590x over previous
import dataclasses
import functools

import jax
import jax.numpy as jnp
from jax import lax
from jax.experimental import pallas as pl
from jax.experimental.pallas import tpu as pltpu
from jax.experimental.pallas import tpu_sc as plsc

NC = 2
NS = 16
NW = NC * NS
K = 256
DEG_W = 16

_mesh = lambda: plsc.VectorSubcoreMesh(core_axis_name="c", subcore_axis_name="s")
_SC_PARAMS = pltpu.CompilerParams(use_tc_tiling_on_sc=False)
_SC_PARAMS_NOLAYOUT = dataclasses.replace(_SC_PARAMS, needs_layout_passes=False)


def _sc_degree(cols3, n_pad, cpw):
    @functools.partial(
        pl.kernel,
        out_type=jax.ShapeDtypeStruct((NW, n_pad), jnp.float32),
        mesh=_mesh(),
        compiler_params=_SC_PARAMS_NOLAYOUT,
        scratch_types=[
            pltpu.VMEM((cpw, K), jnp.int32),
            pltpu.VMEM((n_pad,), jnp.float32),
        ],
    )
    def kern(cols_hbm, out_hbm, cidx, hist):
        c = lax.axis_index("c")
        s = lax.axis_index("s")
        wid = s * NC + c
        pltpu.sync_copy(cols_hbm.at[wid], cidx)
        zeros = jnp.zeros((16,), jnp.float32)
        ones = jnp.ones((16,), jnp.float32)

        @pl.loop(0, n_pad // 16)
        def _(i):
            hist[pl.ds(i * 16, 16)] = zeros

        @pl.loop(0, cpw)
        def _(i):
            @pl.loop(0, K // 16)
            def _(j):
                idx = cidx[i, pl.ds(j * 16, 16)]
                plsc.addupdate_scatter(hist, [idx], ones)

        pltpu.sync_copy(hist, out_hbm.at[wid])

    return kern(cols3)


def _sc_agg_quarters(vals4, rows_s, cols_s, n_pad, dq, nch):
    rpw = n_pad // NS
    nzc, rem = divmod(rpw, K)
    assert nch % 8 == 0

    @functools.partial(
        pl.kernel,
        out_type=jax.ShapeDtypeStruct((4 * NC, n_pad, dq), jnp.float32),
        mesh=_mesh(),
        compiler_params=_SC_PARAMS,
        scratch_types=[
            pltpu.VMEM((nch, K), jnp.int32),
            pltpu.VMEM((nch, K), jnp.int32),
        ] + [pltpu.VMEM((K, dq), jnp.float32) for _ in range(8)] + [
            pltpu.VMEM_SHARED((n_pad, dq), jnp.float32),
            pltpu.VMEM_SHARED((n_pad, dq), jnp.float32),
            pltpu.SemaphoreType.DMA,
            pltpu.SemaphoreType.DMA,
        ],
    )
    def kern(vals_hbm, rows_hbm, cols_hbm, z_hbm, out_hbm,
             ridx, cidx, b0, b1, b2, b3, b4, b5, b6, b7, acc, src_sh,
             gsem, ssem):
        c = lax.axis_index("c")
        s = lax.axis_index("s")
        bufs = (b0, b1, b2, b3, b4, b5, b6, b7)
        n_rows = vals_hbm.shape[1]
        lo = n_rows - (NS - 1) * rpw
        hi = rpw - lo
        pltpu.sync_copy(rows_hbm.at[s], ridx)
        pltpu.sync_copy(cols_hbm.at[s], cidx)
        base = s * rpw

        @pl.loop(0, 4)
        def _(phase):
            piece = 4 * c + phase
            pltpu.sync_copy(vals_hbm.at[piece, pl.ds(base, lo)],
                            src_sh.at[pl.ds(base, lo)])

            @pl.when(s < NS - 1)
            def _():
                pltpu.sync_copy(
                    vals_hbm.at[piece, pl.ds(base + lo, hi)],
                    src_sh.at[pl.ds(base + lo, hi)])

            src = src_sh
            pltpu.sync_copy(z_hbm, b0)

            @pl.loop(0, nzc)
            def _(i):
                pltpu.sync_copy(b0, acc.at[pl.ds(base + i * K, K)])

            if rem:
                pltpu.sync_copy(b0.at[pl.ds(0, rem)],
                                acc.at[pl.ds(base + nzc * K, rem)])
            plsc.subcore_barrier()
            for b in range(8):
                pltpu.async_copy(src.at[ridx.at[b]], bufs[b], gsem)

            def do_group(group, it):
                i0 = 8 * it + 4 * group
                gbufs = bufs[4 * group:4 * group + 4]
                for b in range(4):
                    pltpu.make_async_copy(src.at[ridx.at[i0 + b]],
                                          gbufs[b], gsem).wait()
                scat = [pltpu.async_copy(gbufs[b], acc.at[cidx.at[i0 + b]],
                                         ssem, add=True) for b in range(4)]
                for dsc in scat:
                    dsc.wait()
                nxt = i0 + 8

                @pl.when(nxt < nch)
                def _():
                    for b in range(4):
                        pltpu.async_copy(src.at[ridx.at[nxt + b]],
                                         gbufs[b], gsem)

            @pl.loop(0, nch // 8)
            def _(it):
                do_group(0, it)
                do_group(1, it)

            plsc.subcore_barrier()
            pltpu.sync_copy(acc.at[pl.ds(base, rpw)],
                            out_hbm.at[piece, pl.ds(base, rpw)])

    zeros = jnp.zeros((K, dq), jnp.float32)
    return kern(vals4, rows_s, cols_s, zeros)


def _sc_agg(vals2, rows_s, cols_s, n_pad, d, nch, by_core_half):
    rpw = n_pad // NS
    nzc, rem = divmod(rpw, K)
    assert nch % 8 == 0

    @functools.partial(
        pl.kernel,
        out_type=jax.ShapeDtypeStruct((NC, n_pad, d), jnp.float32),
        mesh=_mesh(),
        compiler_params=_SC_PARAMS,
        scratch_types=[
            pltpu.VMEM((nch, K), jnp.int32),
            pltpu.VMEM((nch, K), jnp.int32),
        ] + [pltpu.VMEM((K, d), jnp.float32) for _ in range(8)] + [
            pltpu.VMEM_SHARED((n_pad, d), jnp.float32),
            pltpu.SemaphoreType.DMA,
            pltpu.SemaphoreType.DMA,
        ],
    )
    def kern(vals_hbm, rows_hbm, cols_hbm, z_hbm, out_hbm,
             ridx, cidx, b0, b1, b2, b3, b4, b5, b6, b7, acc, gsem, ssem):
        c = lax.axis_index("c")
        s = lax.axis_index("s")
        slab = s if by_core_half else s * NC + c
        src = vals_hbm.at[c]
        bufs = (b0, b1, b2, b3, b4, b5, b6, b7)
        pltpu.sync_copy(rows_hbm.at[slab], ridx)
        pltpu.sync_copy(cols_hbm.at[slab], cidx)
        pltpu.sync_copy(z_hbm, b0)

        base = s * rpw

        @pl.loop(0, nzc)
        def _(i):
            pltpu.sync_copy(b0, acc.at[pl.ds(base + i * K, K)])

        if rem:
            pltpu.sync_copy(b0.at[pl.ds(0, rem)], acc.at[pl.ds(base + nzc * K, rem)])
        for b in range(8):
            pltpu.async_copy(src.at[ridx.at[b]], bufs[b], gsem)
        plsc.subcore_barrier()

        def do_group(group, it):
            i0 = 8 * it + 4 * group
            gbufs = bufs[4 * group:4 * group + 4]
            for b in range(4):
                pltpu.make_async_copy(src.at[ridx.at[i0 + b]], gbufs[b], gsem).wait()
            scat = [pltpu.async_copy(gbufs[b], acc.at[cidx.at[i0 + b]], ssem,
                                     add=True) for b in range(4)]
            for dsc in scat:
                dsc.wait()
            nxt = i0 + 8

            @pl.when(nxt < nch)
            def _():
                for b in range(4):
                    pltpu.async_copy(src.at[ridx.at[nxt + b]], gbufs[b], gsem)

        @pl.loop(0, nch // 8)
        def _(it):
            do_group(0, it)
            do_group(1, it)

        plsc.subcore_barrier()
        pltpu.sync_copy(acc.at[pl.ds(base, rpw)], out_hbm.at[c, pl.ds(base, rpw)])

    zeros = jnp.zeros((K, d), jnp.float32)
    return kern(vals2, rows_s, cols_s, zeros)


def _tc_scale(x, deg_m, tm):
    n, d = x.shape
    dq = d // 8

    def body(x_ref, dm_ref, o_ref, dc_ref):
        dsum = jnp.sum(dm_ref[...], axis=0, keepdims=True)
        dinv = jnp.transpose(lax.rsqrt(dsum + 1.0))
        dc_ref[...] = dinv
        xs = dinv * x_ref[...]
        for q in range(8):
            o_ref[q] = xs[:, q * dq:(q + 1) * dq]

    return pl.pallas_call(
        body,
        out_shape=[
            jax.ShapeDtypeStruct((8, n, dq), jnp.float32),
            jax.ShapeDtypeStruct((n, 1), jnp.float32),
        ],
    )(x, deg_m)


def _tc_dense(pq, x, dinvc, W1, b1, W2p, tm):
    n, d_in = x.shape
    dq = d_in // 8
    d_h = W1.shape[1]
    d_o = W2p.shape[1]

    def body(pq_ref, x_ref, dc_ref,
             w1_ref, b1_ref, w2_ref, h_ref, ts_ref):
        dinv = dc_ref[...]
        d2 = dinv * dinv
        x_blk = x_ref[...]
        w1 = w1_ref[...]
        h = b1_ref[...]
        for q in range(8):
            agg_q = (dinv * pq_ref[q]
                     + d2 * x_blk[:, q * dq:(q + 1) * dq])
            h = h + jnp.dot(agg_q, w1[q * dq:(q + 1) * dq],
                            preferred_element_type=jnp.float32)
        h = jnp.maximum(h, 0.0)
        h_ref[...] = h
        t = jnp.dot(h, w2_ref[...], preferred_element_type=jnp.float32)
        tsv = dinv * t
        ts_ref[0] = tsv[:, :d_o // 2]
        ts_ref[1] = tsv[:, d_o // 2:]

    return pl.pallas_call(
        body,
        grid=(n // tm,),
        in_specs=[
            pl.BlockSpec((8, tm, dq), lambda i: (0, i, 0)),
            pl.BlockSpec((tm, d_in), lambda i: (i, 0)),
            pl.BlockSpec((tm, 1), lambda i: (i, 0)),
            pl.BlockSpec((d_in, d_h), lambda i: (0, 0)),
            pl.BlockSpec((1, d_h), lambda i: (0, 0)),
            pl.BlockSpec((d_h, d_o), lambda i: (0, 0)),
        ],
        out_specs=[
            pl.BlockSpec((tm, d_h), lambda i: (i, 0)),
            pl.BlockSpec((2, tm, d_o // 2), lambda i: (0, i, 0)),
        ],
        out_shape=[
            jax.ShapeDtypeStruct((n, d_h), jnp.float32),
            jax.ShapeDtypeStruct((2, n, d_o // 2), jnp.float32),
        ],
    )(pq, x, dinvc, W1, b1, W2p)


def _tc_head(q2, ts2, dinvc, b2p, tm):
    n = ts2.shape[1]
    dh = ts2.shape[2]

    def body(q_ref, ts_ref, dc_ref, b2_ref, o_ref):
        dinv = dc_ref[...]
        for half in range(2):
            z = (dinv * (q_ref[half] + ts_ref[half])
                 + b2_ref[:, half * dh:(half + 1) * dh])
            o_ref[:, half * dh:(half + 1) * dh] = (
                jnp.maximum(z, 0.0) + jnp.log1p(jnp.exp(-jnp.abs(z))))

    return pl.pallas_call(
        body,
        grid=(n // tm,),
        in_specs=[
            pl.BlockSpec((2, tm, dh), lambda i: (0, i, 0)),
            pl.BlockSpec((2, tm, dh), lambda i: (0, i, 0)),
            pl.BlockSpec((tm, 1), lambda i: (i, 0)),
            pl.BlockSpec((1, 2 * dh), lambda i: (0, 0)),
        ],
        out_specs=pl.BlockSpec((tm, 2 * dh), lambda i: (i, 0)),
        out_shape=jax.ShapeDtypeStruct((n, 2 * dh), jnp.float32),
    )(q2, ts2, dinvc, b2p)


def kernel(x, edge_index, W1, b1, W2, b2):
    n = x.shape[0]
    e = edge_index.shape[1]

    n_pad = 8 * NS * -(-(n + DEG_W) // (8 * NS))
    cpw = 8 * -(-e // (NW * K * 8))
    e_pad = NW * cpw * K

    row = edge_index[0].astype(jnp.int32)
    col = edge_index[1].astype(jnp.int32)
    pad = e_pad - e
    prow = jnp.zeros((pad,), jnp.int32)
    pcol = n + (jnp.arange(pad, dtype=jnp.int32) % (n_pad - n))
    rows3 = jnp.concatenate([row, prow]).reshape(NW, cpw, K)
    cols3 = jnp.concatenate([col, pcol]).reshape(NW, cpw, K)

    tm = 2000 if n % 2000 == 0 else 8 * (n // 8)

    deg_m = _sc_degree(cols3, n_pad, cpw)

    xs8, dinvc = _tc_scale(x, deg_m[:, :n], tm)
    dq = x.shape[1] // 8
    cpw2 = 2 * cpw
    rows3s = jnp.concatenate([row, prow]).reshape(NS, cpw2, K)
    cols3s = jnp.concatenate([col, pcol]).reshape(NS, cpw2, K)
    pquart = _sc_agg_quarters(xs8, rows3s, cols3s, n_pad, dq, cpw2)

    wpad = -W2.shape[1] % (4 * dq)
    W2p = jnp.pad(W2, ((0, 0), (0, wpad)))
    b2p = jnp.pad(b2, (0, wpad)).reshape(1, -1)
    h, ts2 = _tc_dense(pquart[:, :n], x, dinvc,
                       b1=b1.reshape(1, -1), W1=W1, W2p=W2p, tm=tm)

    q2 = _sc_agg(ts2, rows3s, cols3s, n_pad, W2p.shape[1] // 2, cpw2,
                 by_core_half=True)
    ev = _tc_head(q2[:, :n], ts2, dinvc, b2p, tm)
    return ev[:, : W2.shape[1]], h


# speedup vs baseline: 32.0670x; 1.5720x over previous
import dataclasses
import functools

import jax
import jax.numpy as jnp
from jax import lax
from jax.experimental import pallas as pl
from jax.experimental.pallas import tpu as pltpu
from jax.experimental.pallas import tpu_sc as plsc

NC = 2
NS = 16
NW = NC * NS
K = 256
DQ = 16

_mesh = lambda: plsc.VectorSubcoreMesh(core_axis_name="c", subcore_axis_name="s")
_SC_PARAMS = pltpu.CompilerParams(use_tc_tiling_on_sc=False)
_SC_PARAMS_NOLAYOUT = dataclasses.replace(_SC_PARAMS, needs_layout_passes=False)


def _sc_degree(cols3, n_pad, cpw):
    @functools.partial(
        pl.kernel,
        out_type=jax.ShapeDtypeStruct((NW, n_pad), jnp.float32),
        mesh=_mesh(),
        compiler_params=_SC_PARAMS_NOLAYOUT,
        scratch_types=[
            pltpu.VMEM((cpw, K), jnp.int32),
            pltpu.VMEM((n_pad,), jnp.float32),
        ],
    )
    def kern(cols_hbm, out_hbm, cidx, hist):
        c = lax.axis_index("c")
        s = lax.axis_index("s")
        wid = s * NC + c
        pltpu.sync_copy(cols_hbm.at[wid], cidx)
        zeros = jnp.zeros((16,), jnp.float32)
        ones = jnp.ones((16,), jnp.float32)

        @pl.loop(0, n_pad // 16)
        def _(i):
            hist[pl.ds(i * 16, 16)] = zeros

        @pl.loop(0, cpw)
        def _(i):
            @pl.loop(0, K // 16)
            def _(j):
                idx = cidx[i, pl.ds(j * 16, 16)]
                plsc.addupdate_scatter(hist, [idx], ones)

        pltpu.sync_copy(hist, out_hbm.at[wid])

    return kern(cols3)


def _sc_agg_pieces(vals, rows_s, cols_s, n_pad, nch, npieces):
    rpw = n_pad // NS
    nzc, rem = divmod(rpw, K)
    phases = npieces // NC
    assert nch % 8 == 0

    @functools.partial(
        pl.kernel,
        out_type=jax.ShapeDtypeStruct((n_pad, 128), jnp.float32),
        mesh=_mesh(),
        compiler_params=_SC_PARAMS,
        scratch_types=[
            pltpu.VMEM((nch, K), jnp.int32),
            pltpu.VMEM((nch, K), jnp.int32),
        ] + [pltpu.VMEM((K, DQ), jnp.float32) for _ in range(8)] + [
            pltpu.VMEM_SHARED((n_pad, DQ), jnp.float32),
            pltpu.VMEM_SHARED((n_pad, DQ), jnp.float32),
            pltpu.SemaphoreType.DMA,
            pltpu.SemaphoreType.DMA,
        ],
    )
    def kern(vals_hbm, rows_hbm, cols_hbm, z_hbm, out_hbm,
             ridx, cidx, b0, b1, b2, b3, b4, b5, b6, b7, src_sh, acc,
             gsem, ssem):
        c = lax.axis_index("c")
        s = lax.axis_index("s")
        bufs = (b0, b1, b2, b3, b4, b5, b6, b7)
        n_rows = vals_hbm.shape[0]
        lo = n_rows - (NS - 1) * rpw
        hi = rpw - lo
        pltpu.sync_copy(rows_hbm.at[s], ridx)
        pltpu.sync_copy(cols_hbm.at[s], cidx)
        base = s * rpw

        @pl.loop(0, phases)
        def _(phase):
            coff = DQ * (phases * c + phase)
            pltpu.sync_copy(vals_hbm.at[pl.ds(base, lo), pl.ds(coff, DQ)],
                            src_sh.at[pl.ds(base, lo)])

            @pl.when(s < NS - 1)
            def _():
                pltpu.sync_copy(
                    vals_hbm.at[pl.ds(base + lo, hi), pl.ds(coff, DQ)],
                    src_sh.at[pl.ds(base + lo, hi)])

            pltpu.sync_copy(z_hbm, b0)

            @pl.loop(0, nzc)
            def _(i):
                pltpu.sync_copy(b0, acc.at[pl.ds(base + i * K, K)])

            if rem:
                pltpu.sync_copy(b0.at[pl.ds(0, rem)],
                                acc.at[pl.ds(base + nzc * K, rem)])
            plsc.subcore_barrier()
            for b in range(8):
                pltpu.async_copy(src_sh.at[ridx.at[b]], bufs[b], gsem)

            def do_group(group, it):
                i0 = 8 * it + 4 * group
                gbufs = bufs[4 * group:4 * group + 4]
                for b in range(4):
                    pltpu.make_async_copy(src_sh.at[ridx.at[i0 + b]],
                                          gbufs[b], gsem).wait()
                scat = [pltpu.async_copy(gbufs[b], acc.at[cidx.at[i0 + b]],
                                         ssem, add=True) for b in range(4)]
                for dsc in scat:
                    dsc.wait()
                nxt = i0 + 8

                @pl.when(nxt < nch)
                def _():
                    for b in range(4):
                        pltpu.async_copy(src_sh.at[ridx.at[nxt + b]],
                                         gbufs[b], gsem)

            @pl.loop(0, nch // 8)
            def _(it):
                do_group(0, it)
                do_group(1, it)

            plsc.subcore_barrier()
            pltpu.sync_copy(acc.at[pl.ds(base, rpw)],
                            out_hbm.at[pl.ds(base, rpw), pl.ds(coff, DQ)])

    zeros = jnp.zeros((K, DQ), jnp.float32)
    return kern(vals, rows_s, cols_s, zeros)


def _tc_scale(x, deg_m):
    n, d = x.shape

    def body(x_ref, dm_ref, o_ref, dc_ref):
        dsum = jnp.sum(dm_ref[...], axis=0, keepdims=True)
        dinv = jnp.transpose(lax.rsqrt(dsum + 1.0))
        dc_ref[...] = dinv
        o_ref[...] = dinv * x_ref[...]

    return pl.pallas_call(
        body,
        out_shape=[
            jax.ShapeDtypeStruct((n, d), jnp.float32),
            jax.ShapeDtypeStruct((n, 1), jnp.float32),
        ],
    )(x, deg_m)


def _tc_dense(p, x, dinvc, W1, b1, W2p, tm):
    n, d_in = x.shape
    d_h = W1.shape[1]
    d_o = W2p.shape[1]

    def body(p_ref, x_ref, dc_ref, w1_ref, b1_ref, w2_ref, h_ref, ts_ref):
        dinv = dc_ref[...]
        agg = dinv * p_ref[...] + (dinv * dinv) * x_ref[...]
        h = jnp.dot(agg, w1_ref[...], preferred_element_type=jnp.float32)
        h = jnp.maximum(h + b1_ref[...], 0.0)
        h_ref[...] = h
        t = jnp.dot(h, w2_ref[...], preferred_element_type=jnp.float32)
        ts_ref[...] = dinv * t

    return pl.pallas_call(
        body,
        grid=(n // tm,),
        in_specs=[
            pl.BlockSpec((tm, d_in), lambda i: (i, 0)),
            pl.BlockSpec((tm, d_in), lambda i: (i, 0)),
            pl.BlockSpec((tm, 1), lambda i: (i, 0)),
            pl.BlockSpec((d_in, d_h), lambda i: (0, 0)),
            pl.BlockSpec((1, d_h), lambda i: (0, 0)),
            pl.BlockSpec((d_h, d_o), lambda i: (0, 0)),
        ],
        out_specs=[
            pl.BlockSpec((tm, d_h), lambda i: (i, 0)),
            pl.BlockSpec((tm, d_o), lambda i: (i, 0)),
        ],
        out_shape=[
            jax.ShapeDtypeStruct((n, d_h), jnp.float32),
            jax.ShapeDtypeStruct((n, d_o), jnp.float32),
        ],
    )(p, x, dinvc, W1, b1, W2p)


def _tc_head(q, ts, dinvc, b2p, tm, d_val):
    n = ts.shape[0]
    d = ts.shape[1]

    def body(q_ref, ts_ref, dc_ref, b2_ref, o_ref):
        dinv = dc_ref[...]
        z = (dinv * (q_ref[...][:, :d_val] + ts_ref[...][:, :d_val])
             + b2_ref[...])
        o_ref[...] = jnp.maximum(z, 0.0) + jnp.log1p(jnp.exp(-jnp.abs(z)))

    return pl.pallas_call(
        body,
        grid=(n // tm,),
        in_specs=[
            pl.BlockSpec((tm, d), lambda i: (i, 0)),
            pl.BlockSpec((tm, d), lambda i: (i, 0)),
            pl.BlockSpec((tm, 1), lambda i: (i, 0)),
            pl.BlockSpec((1, d_val), lambda i: (0, 0)),
        ],
        out_specs=pl.BlockSpec((tm, d_val), lambda i: (i, 0)),
        out_shape=jax.ShapeDtypeStruct((n, d_val), jnp.float32),
    )(q, ts, dinvc, b2p)


def kernel(x, edge_index, W1, b1, W2, b2):
    n = x.shape[0]
    e = edge_index.shape[1]

    n_pad = 8 * NS * -(-(n + DQ) // (8 * NS))
    cpw = 8 * -(-e // (NW * K * 8))
    e_pad = NW * cpw * K

    row = edge_index[0].astype(jnp.int32)
    col = edge_index[1].astype(jnp.int32)
    pad = e_pad - e
    prow = jnp.zeros((pad,), jnp.int32)
    pcol = n + (jnp.arange(pad, dtype=jnp.int32) % (n_pad - n))
    cols3 = jnp.concatenate([col, pcol]).reshape(NW, cpw, K)
    cpw2 = 2 * cpw
    rows3s = jnp.concatenate([row, prow]).reshape(NS, cpw2, K)
    cols3s = jnp.concatenate([col, pcol]).reshape(NS, cpw2, K)

    tm = 2000 if n % 2000 == 0 else 8 * (n // 8)

    deg_m = _sc_degree(cols3, n_pad, cpw)

    xs, dinvc = _tc_scale(x, deg_m[:, :n])
    p = _sc_agg_pieces(xs, rows3s, cols3s, n_pad, cpw2, 8)

    n_cls = W2.shape[1]
    n_agg2 = 4 * DQ
    W2p = jnp.pad(W2, ((0, 0), (0, x.shape[1] - n_cls)))
    b2p = jnp.pad(b2, (0, n_agg2 - n_cls)).reshape(1, -1)
    h, ts = _tc_dense(p, x, dinvc, W1, b1.reshape(1, -1), W2p, tm)

    q = _sc_agg_pieces(ts, rows3s, cols3s, n_pad, cpw2, 4)
    ev = _tc_head(q, ts, dinvc, b2p, tm, n_agg2)
    return ev[:, :n_cls], h


# speedup vs baseline: 32.0754x; 1.0003x over previous
import dataclasses
import functools

import jax
import jax.numpy as jnp
from jax import lax
from jax.experimental import pallas as pl
from jax.experimental.pallas import tpu as pltpu
from jax.experimental.pallas import tpu_sc as plsc

NC = 2
NS = 16
NW = NC * NS
K = 256
DQ = 16

_mesh = lambda: plsc.VectorSubcoreMesh(core_axis_name="c", subcore_axis_name="s")
_SC_PARAMS = pltpu.CompilerParams(use_tc_tiling_on_sc=False)
_SC_PARAMS_NOLAYOUT = dataclasses.replace(_SC_PARAMS, needs_layout_passes=False)


def _sc_degree(cols3, n_pad, cpw):
    @functools.partial(
        pl.kernel,
        out_type=jax.ShapeDtypeStruct((NW, n_pad), jnp.float32),
        mesh=_mesh(),
        compiler_params=_SC_PARAMS_NOLAYOUT,
        scratch_types=[
            pltpu.VMEM((cpw, K), jnp.int32),
            pltpu.VMEM((n_pad,), jnp.float32),
        ],
    )
    def kern(cols_hbm, out_hbm, cidx, hist):
        c = lax.axis_index("c")
        s = lax.axis_index("s")
        wid = s * NC + c
        pltpu.sync_copy(cols_hbm.at[s, pl.ds(c * cpw, cpw)], cidx)
        zeros = jnp.zeros((16,), jnp.float32)
        ones = jnp.ones((16,), jnp.float32)

        @pl.loop(0, n_pad // 16)
        def _(i):
            hist[pl.ds(i * 16, 16)] = zeros

        @pl.loop(0, cpw)
        def _(i):
            @pl.loop(0, K // 16)
            def _(j):
                idx = cidx[i, pl.ds(j * 16, 16)]
                plsc.addupdate_scatter(hist, [idx], ones)

        pltpu.sync_copy(hist, out_hbm.at[wid])

    return kern(cols3)


def _sc_agg_pieces(vals, rows_s, cols_s, n_pad, nch, npieces):
    rpw = n_pad // NS
    nzc, rem = divmod(rpw, K)
    phases = npieces // NC
    assert nch % 8 == 0

    @functools.partial(
        pl.kernel,
        out_type=jax.ShapeDtypeStruct((n_pad, 128), jnp.float32),
        mesh=_mesh(),
        compiler_params=_SC_PARAMS,
        scratch_types=[
            pltpu.VMEM((nch, K), jnp.int32),
            pltpu.VMEM((nch, K), jnp.int32),
        ] + [pltpu.VMEM((K, DQ), jnp.float32) for _ in range(8)] + [
            pltpu.VMEM_SHARED((n_pad, DQ), jnp.float32),
            pltpu.VMEM_SHARED((n_pad, DQ), jnp.float32),
            pltpu.SemaphoreType.DMA,
            pltpu.SemaphoreType.DMA,
        ],
    )
    def kern(vals_hbm, rows_hbm, cols_hbm, z_hbm, out_hbm,
             ridx, cidx, b0, b1, b2, b3, b4, b5, b6, b7, src_sh, acc,
             gsem, ssem):
        c = lax.axis_index("c")
        s = lax.axis_index("s")
        bufs = (b0, b1, b2, b3, b4, b5, b6, b7)
        n_rows = vals_hbm.shape[0]
        lo = n_rows - (NS - 1) * rpw
        hi = rpw - lo
        pltpu.sync_copy(rows_hbm.at[s], ridx)
        pltpu.sync_copy(cols_hbm.at[s], cidx)
        base = s * rpw

        @pl.loop(0, phases)
        def _(phase):
            coff = DQ * (phases * c + phase)
            pltpu.sync_copy(vals_hbm.at[pl.ds(base, lo), pl.ds(coff, DQ)],
                            src_sh.at[pl.ds(base, lo)])

            @pl.when(s < NS - 1)
            def _():
                pltpu.sync_copy(
                    vals_hbm.at[pl.ds(base + lo, hi), pl.ds(coff, DQ)],
                    src_sh.at[pl.ds(base + lo, hi)])

            pltpu.sync_copy(z_hbm, b0)

            @pl.loop(0, nzc)
            def _(i):
                pltpu.sync_copy(b0, acc.at[pl.ds(base + i * K, K)])

            if rem:
                pltpu.sync_copy(b0.at[pl.ds(0, rem)],
                                acc.at[pl.ds(base + nzc * K, rem)])
            plsc.subcore_barrier()
            for b in range(8):
                pltpu.async_copy(src_sh.at[ridx.at[b]], bufs[b], gsem)

            def do_group(group, it):
                i0 = 8 * it + 4 * group
                gbufs = bufs[4 * group:4 * group + 4]
                for b in range(4):
                    pltpu.make_async_copy(src_sh.at[ridx.at[i0 + b]],
                                          gbufs[b], gsem).wait()
                scat = [pltpu.async_copy(gbufs[b], acc.at[cidx.at[i0 + b]],
                                         ssem, add=True) for b in range(4)]
                for dsc in scat:
                    dsc.wait()
                nxt = i0 + 8

                @pl.when(nxt < nch)
                def _():
                    for b in range(4):
                        pltpu.async_copy(src_sh.at[ridx.at[nxt + b]],
                                         gbufs[b], gsem)

            @pl.loop(0, nch // 8)
            def _(it):
                do_group(0, it)
                do_group(1, it)

            plsc.subcore_barrier()
            pltpu.sync_copy(acc.at[pl.ds(base, rpw)],
                            out_hbm.at[pl.ds(base, rpw), pl.ds(coff, DQ)])

    zeros = jnp.zeros((K, DQ), jnp.float32)
    return kern(vals, rows_s, cols_s, zeros)


def _tc_scale(x, deg_m):
    n, d = x.shape

    def body(x_ref, dm_ref, o_ref, dc_ref):
        dsum = jnp.sum(dm_ref[...], axis=0, keepdims=True)
        dinv = jnp.transpose(lax.rsqrt(dsum + 1.0))
        dc_ref[...] = dinv
        o_ref[...] = dinv * x_ref[...]

    return pl.pallas_call(
        body,
        out_shape=[
            jax.ShapeDtypeStruct((n, d), jnp.float32),
            jax.ShapeDtypeStruct((n, 1), jnp.float32),
        ],
    )(x, deg_m)


def _tc_dense(p, x, dinvc, W1, b1, W2p, tm):
    n, d_in = x.shape
    d_h = W1.shape[1]
    d_o = W2p.shape[1]

    def body(p_ref, x_ref, dc_ref, w1_ref, b1_ref, w2_ref, h_ref, ts_ref):
        dinv = dc_ref[...]
        agg = dinv * p_ref[...] + (dinv * dinv) * x_ref[...]
        h = jnp.dot(agg, w1_ref[...], preferred_element_type=jnp.float32)
        h = jnp.maximum(h + b1_ref[...], 0.0)
        h_ref[...] = h
        t = jnp.dot(h, w2_ref[...], preferred_element_type=jnp.float32)
        ts_ref[...] = dinv * t

    return pl.pallas_call(
        body,
        grid=(n // tm,),
        in_specs=[
            pl.BlockSpec((tm, d_in), lambda i: (i, 0)),
            pl.BlockSpec((tm, d_in), lambda i: (i, 0)),
            pl.BlockSpec((tm, 1), lambda i: (i, 0)),
            pl.BlockSpec((d_in, d_h), lambda i: (0, 0)),
            pl.BlockSpec((1, d_h), lambda i: (0, 0)),
            pl.BlockSpec((d_h, d_o), lambda i: (0, 0)),
        ],
        out_specs=[
            pl.BlockSpec((tm, d_h), lambda i: (i, 0)),
            pl.BlockSpec((tm, d_o), lambda i: (i, 0)),
        ],
        out_shape=[
            jax.ShapeDtypeStruct((n, d_h), jnp.float32),
            jax.ShapeDtypeStruct((n, d_o), jnp.float32),
        ],
    )(p, x, dinvc, W1, b1, W2p)


def _tc_head(q, ts, dinvc, b2p, tm):
    n = ts.shape[0]
    d = ts.shape[1]
    d_val = b2p.shape[1]

    def body(q_ref, ts_ref, dc_ref, b2_ref, o_ref):
        dinv = dc_ref[...]
        z = (dinv * (q_ref[...][:, :d_val] + ts_ref[...][:, :d_val])
             + b2_ref[...])
        o_ref[...] = jnp.maximum(z, 0.0) + jnp.log1p(jnp.exp(-jnp.abs(z)))

    return pl.pallas_call(
        body,
        grid=(n // tm,),
        in_specs=[
            pl.BlockSpec((tm, d), lambda i: (i, 0)),
            pl.BlockSpec((tm, d), lambda i: (i, 0)),
            pl.BlockSpec((tm, 1), lambda i: (i, 0)),
            pl.BlockSpec((1, d_val), lambda i: (0, 0)),
        ],
        out_specs=pl.BlockSpec((tm, d_val), lambda i: (i, 0)),
        out_shape=jax.ShapeDtypeStruct((n, d_val), jnp.float32),
    )(q, ts, dinvc, b2p)


def kernel(x, edge_index, W1, b1, W2, b2):
    n = x.shape[0]
    e = edge_index.shape[1]

    n_pad = 8 * NS * -(-(n + DQ) // (8 * NS))
    cpw = 8 * -(-e // (NW * K * 8))
    e_pad = NW * cpw * K

    row = edge_index[0].astype(jnp.int32)
    col = edge_index[1].astype(jnp.int32)
    pad = e_pad - e
    prow = jnp.zeros((pad,), jnp.int32)
    pcol = n + (jnp.arange(pad, dtype=jnp.int32) % (n_pad - n))
    cpw2 = 2 * cpw
    rows3s = jnp.concatenate([row, prow]).reshape(NS, cpw2, K)
    cols3s = jnp.concatenate([col, pcol]).reshape(NS, cpw2, K)

    tm = 2000 if n % 2000 == 0 else 8 * (n // 8)

    deg_m = _sc_degree(cols3s, n_pad, cpw)

    xs, dinvc = _tc_scale(x, deg_m[:, :n])
    p = _sc_agg_pieces(xs, rows3s, cols3s, n_pad, cpw2, 8)

    n_cls = W2.shape[1]
    W2p = jnp.pad(W2, ((0, 0), (0, x.shape[1] - n_cls)))
    h, ts = _tc_dense(p, x, dinvc, W1, b1.reshape(1, -1), W2p, tm)

    q = _sc_agg_pieces(ts, rows3s, cols3s, n_pad, cpw2, 4)
    ev = _tc_head(q, ts, dinvc, b2.reshape(1, -1), tm)
    return ev, h
